# Initial kernel scaffold; baseline (speedup 1.0000x reference)
import jax
import jax.numpy as jnp
from jax.experimental import pallas as pl


def kernel(x, edge_index, params):
    raise NotImplementedError("write your pallas kernel here")


# speedup vs baseline: 10.1487x; 10.1487x over previous
import functools

import jax
import jax.numpy as jnp
from jax import lax
from jax.experimental import pallas as pl
from jax.experimental.pallas import tpu as pltpu
from jax.experimental.pallas import tpu_sc as plsc

N = 10000
NP = 10240
E = 320000
NC = 2
NS = 16
NW = NC * NS
EW = E // NW
K = 80
ROWS_PER_TILE = NP // NS


def _make_scatter(W):
  mesh = plsc.VectorSubcoreMesh(core_axis_name="c", subcore_axis_name="s")

  @functools.partial(
      pl.kernel,
      out_type=jax.ShapeDtypeStruct((NC, NP, W), jnp.float32),
      mesh=mesh,
      scratch_types=[
          pltpu.VMEM((K,), jnp.int32),
          pltpu.VMEM((K,), jnp.int32),
          pltpu.VMEM((K, W), jnp.float32),
          pltpu.VMEM_SHARED((NP, W), jnp.float32),
          pltpu.SemaphoreType.DMA,
      ],
  )
  def scatter_kernel(u_hbm, src_hbm, dst_hbm, zeros_hbm, out_hbm,
                     sidx, didx, rows, acc, sem):
    c = lax.axis_index("c")
    s = lax.axis_index("s")
    stripe = pl.ds(s * ROWS_PER_TILE, ROWS_PER_TILE)
    pltpu.sync_copy(zeros_hbm, acc.at[stripe])
    plsc.subcore_barrier()

    base0 = (c * NS + s) * EW

    def body(i, carry):
      base = base0 + i * K
      pltpu.sync_copy(src_hbm.at[pl.ds(base, K)], sidx)
      pltpu.sync_copy(dst_hbm.at[pl.ds(base, K)], didx)
      pltpu.async_copy(u_hbm.at[sidx], rows, sem).wait()
      pltpu.sync_copy(rows, acc.at[didx], add=True)
      return carry

    lax.fori_loop(0, EW // K, body, 0)
    plsc.subcore_barrier()
    pltpu.sync_copy(acc.at[stripe], out_hbm.at[c, stripe])

  return scatter_kernel


def _make_degree():
  mesh = plsc.VectorSubcoreMesh(core_axis_name="c", subcore_axis_name="s")

  @functools.partial(
      pl.kernel,
      out_type=jax.ShapeDtypeStruct((NC, NP, 128), jnp.float32),
      mesh=mesh,
      scratch_types=[
          pltpu.VMEM((K,), jnp.int32),
          pltpu.VMEM((K, 128), jnp.float32),
          pltpu.VMEM_SHARED((NP, 128), jnp.float32),
          pltpu.SemaphoreType.DMA,
      ],
  )
  def degree_kernel(dst_hbm, zeros_hbm, ones_hbm, out_hbm,
                    didx, ones_v, acc, sem):
    c = lax.axis_index("c")
    s = lax.axis_index("s")
    stripe = pl.ds(s * ROWS_PER_TILE, ROWS_PER_TILE)
    pltpu.sync_copy(zeros_hbm, acc.at[stripe])
    pltpu.sync_copy(ones_hbm, ones_v)
    plsc.subcore_barrier()

    base0 = (c * NS + s) * EW

    def body(i, carry):
      base = base0 + i * K
      pltpu.sync_copy(dst_hbm.at[pl.ds(base, K)], didx)
      pltpu.sync_copy(ones_v, acc.at[didx], add=True)
      return carry

    lax.fori_loop(0, EW // K, body, 0)
    plsc.subcore_barrier()
    pltpu.sync_copy(acc.at[stripe], out_hbm.at[c, stripe])

  return degree_kernel


_scatter128 = _make_scatter(128)
_degree = _make_degree()


def _dis(da_ref, db_ref):
  deg = da_ref[:, 0:1] + db_ref[:, 0:1] + 1.0
  return lax.rsqrt(deg)


def _tc_a(x, w_in, dega, degb, R=1000):
  def body(x_ref, w_ref, da_ref, db_ref, u1_ref):
    dis = _dis(da_ref, db_ref)
    h = jnp.dot(x_ref[...], w_ref[...], preferred_element_type=jnp.float32)
    u1_ref[...] = h * dis

  return pl.pallas_call(
      body,
      grid=(N // R,),
      in_specs=[
          pl.BlockSpec((R, 128), lambda i: (i, 0)),
          pl.BlockSpec((128, 128), lambda i: (0, 0)),
          pl.BlockSpec((R, 128), lambda i: (i, 0)),
          pl.BlockSpec((R, 128), lambda i: (i, 0)),
      ],
      out_specs=pl.BlockSpec((R, 128), lambda i: (i, 0)),
      out_shape=jax.ShapeDtypeStruct((N, 128), jnp.float32),
  )(x, w_in, dega, degb)


def _tc_b(agg1a, agg1b, u1, dega, degb, b_in, w1, R=1000):
  def body(aa_ref, ab_ref, u1_ref, da_ref, db_ref, bin_ref, w1_ref,
           h1_ref, u2_ref):
    dis = _dis(da_ref, db_ref)
    h1 = dis * (aa_ref[...] + ab_ref[...] + u1_ref[...]) + bin_ref[...]
    h1 = jnp.maximum(h1, 0.0)
    h1_ref[...] = h1
    u2_ref[...] = jnp.dot(h1, w1_ref[...],
                          preferred_element_type=jnp.float32) * dis

  return pl.pallas_call(
      body,
      grid=(N // R,),
      in_specs=[
          pl.BlockSpec((R, 128), lambda i: (i, 0)),
          pl.BlockSpec((R, 128), lambda i: (i, 0)),
          pl.BlockSpec((R, 128), lambda i: (i, 0)),
          pl.BlockSpec((R, 128), lambda i: (i, 0)),
          pl.BlockSpec((R, 128), lambda i: (i, 0)),
          pl.BlockSpec((1, 128), lambda i: (0, 0)),
          pl.BlockSpec((128, 128), lambda i: (0, 0)),
      ],
      out_specs=[
          pl.BlockSpec((R, 128), lambda i: (i, 0)),
          pl.BlockSpec((R, 128), lambda i: (i, 0)),
      ],
      out_shape=[
          jax.ShapeDtypeStruct((N, 128), jnp.float32),
          jax.ShapeDtypeStruct((N, 128), jnp.float32),
      ],
  )(agg1a, agg1b, u1, dega, degb, b_in, w1)


def _lstm_cell(g, c_prev):
  i = jax.nn.sigmoid(g[:, 0:128])
  f = jax.nn.sigmoid(g[:, 128:256])
  gg = jnp.tanh(g[:, 256:384])
  o = jax.nn.sigmoid(g[:, 384:512])
  c = f * c_prev + i * gg
  return o * jnp.tanh(c), c


def _tc_c(agg2a, agg2b, u2, h1, dega, degb, b1,
          wihf, whhf, bf, wihb, whhb, bb, wf, wb, w_out, R=400):
  def body(aa_ref, ab_ref, u2_ref, h1_ref, da_ref, db_ref, b1_ref,
           wihf_ref, whhf_ref, bf_ref, wihb_ref, whhb_ref, bb_ref,
           wf_ref, wb_ref, wo_ref, u3_ref):
    dis = _dis(da_ref, db_ref)
    h2 = dis * (aa_ref[...] + ab_ref[...] + u2_ref[...]) + b1_ref[...]
    h2 = jnp.maximum(h2, 0.0)
    h1 = h1_ref[...]

    def mm(a, b):
      return jnp.dot(a, b, preferred_element_type=jnp.float32)

    hf0, cf0 = _lstm_cell(mm(h1, wihf_ref[...]) + bf_ref[...], 0.0)
    hf1, _ = _lstm_cell(mm(h2, wihf_ref[...]) + mm(hf0, whhf_ref[...])
                        + bf_ref[...], cf0)
    hb0, cb0 = _lstm_cell(mm(h2, wihb_ref[...]) + bb_ref[...], 0.0)
    hb1, _ = _lstm_cell(mm(h1, wihb_ref[...]) + mm(hb0, whhb_ref[...])
                        + bb_ref[...], cb0)
    a0 = mm(hf0, wf_ref[...]) + mm(hb1, wb_ref[...])
    a1 = mm(hf1, wf_ref[...]) + mm(hb0, wb_ref[...])
    alpha = jax.nn.sigmoid(a0[:, 0:1] - a1[:, 0:1])
    h3 = alpha * h1 + (1.0 - alpha) * h2
    u3_ref[...] = mm(h3, wo_ref[...]) * dis

  return pl.pallas_call(
      body,
      grid=(N // R,),
      in_specs=[
          pl.BlockSpec((R, 128), lambda i: (i, 0)),
          pl.BlockSpec((R, 128), lambda i: (i, 0)),
          pl.BlockSpec((R, 128), lambda i: (i, 0)),
          pl.BlockSpec((R, 128), lambda i: (i, 0)),
          pl.BlockSpec((R, 128), lambda i: (i, 0)),
          pl.BlockSpec((R, 128), lambda i: (i, 0)),
          pl.BlockSpec((1, 128), lambda i: (0, 0)),
          pl.BlockSpec((128, 512), lambda i: (0, 0)),
          pl.BlockSpec((128, 512), lambda i: (0, 0)),
          pl.BlockSpec((1, 512), lambda i: (0, 0)),
          pl.BlockSpec((128, 512), lambda i: (0, 0)),
          pl.BlockSpec((128, 512), lambda i: (0, 0)),
          pl.BlockSpec((1, 512), lambda i: (0, 0)),
          pl.BlockSpec((128, 8), lambda i: (0, 0)),
          pl.BlockSpec((128, 8), lambda i: (0, 0)),
          pl.BlockSpec((128, 128), lambda i: (0, 0)),
      ],
      out_specs=pl.BlockSpec((R, 128), lambda i: (i, 0)),
      out_shape=jax.ShapeDtypeStruct((N, 128), jnp.float32),
  )(agg2a, agg2b, u2, h1, dega, degb, b1,
    wihf, whhf, bf, wihb, whhb, bb, wf, wb, w_out)


def _tc_d(agg3a, agg3b, u3, dega, degb, b_out, R=1000):
  def body(aa_ref, ab_ref, u3_ref, da_ref, db_ref, bo_ref, out_ref):
    dis = _dis(da_ref, db_ref)
    full = dis * (aa_ref[...] + ab_ref[...] + u3_ref[...])
    out_ref[...] = full[:, 0:40] + bo_ref[...]

  return pl.pallas_call(
      body,
      grid=(N // R,),
      in_specs=[
          pl.BlockSpec((R, 128), lambda i: (i, 0)),
          pl.BlockSpec((R, 128), lambda i: (i, 0)),
          pl.BlockSpec((R, 128), lambda i: (i, 0)),
          pl.BlockSpec((R, 128), lambda i: (i, 0)),
          pl.BlockSpec((R, 128), lambda i: (i, 0)),
          pl.BlockSpec((1, 40), lambda i: (0, 0)),
      ],
      out_specs=pl.BlockSpec((R, 40), lambda i: (i, 0)),
      out_shape=jax.ShapeDtypeStruct((N, 40), jnp.float32),
  )(agg3a, agg3b, u3, dega, degb, b_out)


@jax.jit
def kernel(x, edge_index, params):
  src = edge_index[0]
  dst = edge_index[1]

  w_in = params["in_gc"]["W"]
  b_in = params["in_gc"]["b"][None, :]
  w1 = params["gc1"]["W"]
  b1 = params["gc1"]["b"][None, :]
  w_out = jnp.pad(params["out_gc"]["W"], ((0, 0), (0, 88)))
  b_out = params["out_gc"]["b"][None, :]
  lp = params["out_jk"]["lstm"]
  wihf = lp["Wih_f"].T
  whhf = lp["Whh_f"].T
  bf = (lp["bih_f"] + lp["bhh_f"])[None, :]
  wihb = lp["Wih_b"].T
  whhb = lp["Whh_b"].T
  bb = (lp["bih_b"] + lp["bhh_b"])[None, :]
  att = params["out_jk"]["att_W"]
  wf = jnp.pad(att[:, :128].T, ((0, 0), (0, 7)))
  wb = jnp.pad(att[:, 128:].T, ((0, 0), (0, 7)))

  ones128 = jnp.ones((K, 128), jnp.float32)
  zeros128 = jnp.zeros((ROWS_PER_TILE, 128), jnp.float32)

  deg = _degree(dst, zeros128, ones128)
  dega, degb = deg[0], deg[1]

  u1 = _tc_a(x, w_in, dega, degb)
  agg1 = _scatter128(u1, src, dst, zeros128)
  h1, u2 = _tc_b(agg1[0], agg1[1], u1, dega, degb, b_in, w1)
  agg2 = _scatter128(u2, src, dst, zeros128)
  u3 = _tc_c(agg2[0], agg2[1], u2, h1, dega, degb, b1,
             wihf, whhf, bf, wihb, whhb, bb, wf, wb, w_out)
  agg3 = _scatter128(u3, src, dst, zeros128)
  return _tc_d(agg3[0], agg3[1], u3, dega, degb, b_out)


# speedup vs baseline: 14.7457x; 1.4530x over previous
import functools

import jax
import jax.numpy as jnp
from jax import lax
from jax.experimental import pallas as pl
from jax.experimental.pallas import tpu as pltpu
from jax.experimental.pallas import tpu_sc as plsc

N = 10000
NP = 10240
E = 320000
NC = 2
NS = 16
NW = NC * NS
EW = E // NW
K = 80
ROWS_PER_TILE = NP // NS


def _make_scatter(W):
  mesh = plsc.VectorSubcoreMesh(core_axis_name="c", subcore_axis_name="s")

  @functools.partial(
      pl.kernel,
      out_type=jax.ShapeDtypeStruct((NC, NP, W), jnp.float32),
      mesh=mesh,
      scratch_types=[
          pltpu.VMEM((K,), jnp.int32),
          pltpu.VMEM((K,), jnp.int32),
          pltpu.VMEM((K, W), jnp.float32),
          pltpu.VMEM((K,), jnp.int32),
          pltpu.VMEM((K,), jnp.int32),
          pltpu.VMEM((K, W), jnp.float32),
          pltpu.VMEM_SHARED((NP, W), jnp.float32),
          pltpu.SemaphoreType.DMA,
          pltpu.SemaphoreType.DMA,
      ],
  )
  def scatter_kernel(u_hbm, src_hbm, dst_hbm, zeros_hbm, out_hbm,
                     sidx_a, didx_a, rows_a, sidx_b, didx_b, rows_b,
                     acc, sem_a, sem_b):
    c = lax.axis_index("c")
    s = lax.axis_index("s")
    stripe = pl.ds(s * ROWS_PER_TILE, ROWS_PER_TILE)
    pltpu.sync_copy(zeros_hbm, acc.at[stripe])
    plsc.subcore_barrier()

    base0 = (c * NS + s) * EW
    n_iters = EW // K

    def load_idx(base, sidx, didx):
      pltpu.sync_copy(src_hbm.at[pl.ds(base, K)], sidx)
      pltpu.sync_copy(dst_hbm.at[pl.ds(base, K)], didx)

    load_idx(base0, sidx_a, didx_a)
    pltpu.async_copy(u_hbm.at[sidx_a], rows_a, sem_a)
    load_idx(base0 + K, sidx_b, didx_b)
    pltpu.async_copy(u_hbm.at[sidx_b], rows_b, sem_b)

    def body(j, carry):
      pltpu.make_async_copy(u_hbm.at[sidx_a], rows_a, sem_a).wait()
      pltpu.sync_copy(rows_a, acc.at[didx_a], add=True)

      @pl.when(j < 61)
      def _():
        load_idx(base0 + (2 * j + 2) * K, sidx_a, didx_a)
        pltpu.async_copy(u_hbm.at[sidx_a], rows_a, sem_a)

      pltpu.make_async_copy(u_hbm.at[sidx_b], rows_b, sem_b).wait()
      pltpu.sync_copy(rows_b, acc.at[didx_b], add=True)

      @pl.when(j < 61)
      def _():
        load_idx(base0 + (2 * j + 3) * K, sidx_b, didx_b)
        pltpu.async_copy(u_hbm.at[sidx_b], rows_b, sem_b)

      return carry

    lax.fori_loop(0, 62, body, 0)
    load_idx(base0 + (n_iters - 1) * K, sidx_a, didx_a)
    pltpu.async_copy(u_hbm.at[sidx_a], rows_a, sem_a).wait()
    pltpu.sync_copy(rows_a, acc.at[didx_a], add=True)

    plsc.subcore_barrier()
    pltpu.sync_copy(acc.at[stripe], out_hbm.at[c, stripe])

  return scatter_kernel


def _make_degree():
  mesh = plsc.VectorSubcoreMesh(core_axis_name="c", subcore_axis_name="s")

  @functools.partial(
      pl.kernel,
      out_type=jax.ShapeDtypeStruct((NC, NP, 128), jnp.float32),
      mesh=mesh,
      scratch_types=[
          pltpu.VMEM((K,), jnp.int32),
          pltpu.VMEM((K, 128), jnp.float32),
          pltpu.VMEM_SHARED((NP, 128), jnp.float32),
          pltpu.SemaphoreType.DMA,
      ],
  )
  def degree_kernel(dst_hbm, zeros_hbm, ones_hbm, out_hbm,
                    didx, ones_v, acc, sem):
    c = lax.axis_index("c")
    s = lax.axis_index("s")
    stripe = pl.ds(s * ROWS_PER_TILE, ROWS_PER_TILE)
    pltpu.sync_copy(zeros_hbm, acc.at[stripe])
    pltpu.sync_copy(ones_hbm, ones_v)
    plsc.subcore_barrier()

    base0 = (c * NS + s) * EW

    def body(i, carry):
      base = base0 + i * K
      pltpu.sync_copy(dst_hbm.at[pl.ds(base, K)], didx)
      pltpu.sync_copy(ones_v, acc.at[didx], add=True)
      return carry

    lax.fori_loop(0, EW // K, body, 0)
    plsc.subcore_barrier()
    pltpu.sync_copy(acc.at[stripe], out_hbm.at[c, stripe])

  return degree_kernel


_scatter128 = _make_scatter(128)
_degree = _make_degree()


def _dis(da_ref, db_ref):
  deg = da_ref[:, 0:1] + db_ref[:, 0:1] + 1.0
  return lax.rsqrt(deg)


def _tc_a(x, w_in, dega, degb, R=1000):
  def body(x_ref, w_ref, da_ref, db_ref, u1_ref):
    dis = _dis(da_ref, db_ref)
    h = jnp.dot(x_ref[...], w_ref[...], preferred_element_type=jnp.float32)
    u1_ref[...] = h * dis

  return pl.pallas_call(
      body,
      grid=(N // R,),
      in_specs=[
          pl.BlockSpec((R, 128), lambda i: (i, 0)),
          pl.BlockSpec((128, 128), lambda i: (0, 0)),
          pl.BlockSpec((R, 128), lambda i: (i, 0)),
          pl.BlockSpec((R, 128), lambda i: (i, 0)),
      ],
      out_specs=pl.BlockSpec((R, 128), lambda i: (i, 0)),
      out_shape=jax.ShapeDtypeStruct((N, 128), jnp.float32),
  )(x, w_in, dega, degb)


def _tc_b(agg1a, agg1b, u1, dega, degb, b_in, w1, R=1000):
  def body(aa_ref, ab_ref, u1_ref, da_ref, db_ref, bin_ref, w1_ref,
           h1_ref, u2_ref):
    dis = _dis(da_ref, db_ref)
    h1 = dis * (aa_ref[...] + ab_ref[...] + u1_ref[...]) + bin_ref[...]
    h1 = jnp.maximum(h1, 0.0)
    h1_ref[...] = h1
    u2_ref[...] = jnp.dot(h1, w1_ref[...],
                          preferred_element_type=jnp.float32) * dis

  return pl.pallas_call(
      body,
      grid=(N // R,),
      in_specs=[
          pl.BlockSpec((R, 128), lambda i: (i, 0)),
          pl.BlockSpec((R, 128), lambda i: (i, 0)),
          pl.BlockSpec((R, 128), lambda i: (i, 0)),
          pl.BlockSpec((R, 128), lambda i: (i, 0)),
          pl.BlockSpec((R, 128), lambda i: (i, 0)),
          pl.BlockSpec((1, 128), lambda i: (0, 0)),
          pl.BlockSpec((128, 128), lambda i: (0, 0)),
      ],
      out_specs=[
          pl.BlockSpec((R, 128), lambda i: (i, 0)),
          pl.BlockSpec((R, 128), lambda i: (i, 0)),
      ],
      out_shape=[
          jax.ShapeDtypeStruct((N, 128), jnp.float32),
          jax.ShapeDtypeStruct((N, 128), jnp.float32),
      ],
  )(agg1a, agg1b, u1, dega, degb, b_in, w1)


def _lstm_cell(g, c_prev):
  i = jax.nn.sigmoid(g[:, 0:128])
  f = jax.nn.sigmoid(g[:, 128:256])
  gg = jnp.tanh(g[:, 256:384])
  o = jax.nn.sigmoid(g[:, 384:512])
  c = f * c_prev + i * gg
  return o * jnp.tanh(c), c


def _tc_c(agg2a, agg2b, u2, h1, dega, degb, b1,
          wihf, whhf, bf, wihb, whhb, bb, wf, wb, w_out, R=400):
  def body(aa_ref, ab_ref, u2_ref, h1_ref, da_ref, db_ref, b1_ref,
           wihf_ref, whhf_ref, bf_ref, wihb_ref, whhb_ref, bb_ref,
           wf_ref, wb_ref, wo_ref, u3_ref):
    dis = _dis(da_ref, db_ref)
    h2 = dis * (aa_ref[...] + ab_ref[...] + u2_ref[...]) + b1_ref[...]
    h2 = jnp.maximum(h2, 0.0)
    h1 = h1_ref[...]

    def mm(a, b):
      return jnp.dot(a, b, preferred_element_type=jnp.float32)

    hf0, cf0 = _lstm_cell(mm(h1, wihf_ref[...]) + bf_ref[...], 0.0)
    hf1, _ = _lstm_cell(mm(h2, wihf_ref[...]) + mm(hf0, whhf_ref[...])
                        + bf_ref[...], cf0)
    hb0, cb0 = _lstm_cell(mm(h2, wihb_ref[...]) + bb_ref[...], 0.0)
    hb1, _ = _lstm_cell(mm(h1, wihb_ref[...]) + mm(hb0, whhb_ref[...])
                        + bb_ref[...], cb0)
    a0 = mm(hf0, wf_ref[...]) + mm(hb1, wb_ref[...])
    a1 = mm(hf1, wf_ref[...]) + mm(hb0, wb_ref[...])
    alpha = jax.nn.sigmoid(a0[:, 0:1] - a1[:, 0:1])
    h3 = alpha * h1 + (1.0 - alpha) * h2
    u3_ref[...] = mm(h3, wo_ref[...]) * dis

  return pl.pallas_call(
      body,
      grid=(N // R,),
      in_specs=[
          pl.BlockSpec((R, 128), lambda i: (i, 0)),
          pl.BlockSpec((R, 128), lambda i: (i, 0)),
          pl.BlockSpec((R, 128), lambda i: (i, 0)),
          pl.BlockSpec((R, 128), lambda i: (i, 0)),
          pl.BlockSpec((R, 128), lambda i: (i, 0)),
          pl.BlockSpec((R, 128), lambda i: (i, 0)),
          pl.BlockSpec((1, 128), lambda i: (0, 0)),
          pl.BlockSpec((128, 512), lambda i: (0, 0)),
          pl.BlockSpec((128, 512), lambda i: (0, 0)),
          pl.BlockSpec((1, 512), lambda i: (0, 0)),
          pl.BlockSpec((128, 512), lambda i: (0, 0)),
          pl.BlockSpec((128, 512), lambda i: (0, 0)),
          pl.BlockSpec((1, 512), lambda i: (0, 0)),
          pl.BlockSpec((128, 8), lambda i: (0, 0)),
          pl.BlockSpec((128, 8), lambda i: (0, 0)),
          pl.BlockSpec((128, 128), lambda i: (0, 0)),
      ],
      out_specs=pl.BlockSpec((R, 128), lambda i: (i, 0)),
      out_shape=jax.ShapeDtypeStruct((N, 128), jnp.float32),
  )(agg2a, agg2b, u2, h1, dega, degb, b1,
    wihf, whhf, bf, wihb, whhb, bb, wf, wb, w_out)


def _tc_d(agg3a, agg3b, u3, dega, degb, b_out, R=1000):
  def body(aa_ref, ab_ref, u3_ref, da_ref, db_ref, bo_ref, out_ref):
    dis = _dis(da_ref, db_ref)
    full = dis * (aa_ref[...] + ab_ref[...] + u3_ref[...])
    out_ref[...] = full[:, 0:40] + bo_ref[...]

  return pl.pallas_call(
      body,
      grid=(N // R,),
      in_specs=[
          pl.BlockSpec((R, 128), lambda i: (i, 0)),
          pl.BlockSpec((R, 128), lambda i: (i, 0)),
          pl.BlockSpec((R, 128), lambda i: (i, 0)),
          pl.BlockSpec((R, 128), lambda i: (i, 0)),
          pl.BlockSpec((R, 128), lambda i: (i, 0)),
          pl.BlockSpec((1, 40), lambda i: (0, 0)),
      ],
      out_specs=pl.BlockSpec((R, 40), lambda i: (i, 0)),
      out_shape=jax.ShapeDtypeStruct((N, 40), jnp.float32),
  )(agg3a, agg3b, u3, dega, degb, b_out)


@jax.jit
def kernel(x, edge_index, params):
  src = edge_index[0]
  dst = edge_index[1]

  w_in = params["in_gc"]["W"]
  b_in = params["in_gc"]["b"][None, :]
  w1 = params["gc1"]["W"]
  b1 = params["gc1"]["b"][None, :]
  w_out = jnp.pad(params["out_gc"]["W"], ((0, 0), (0, 88)))
  b_out = params["out_gc"]["b"][None, :]
  lp = params["out_jk"]["lstm"]
  wihf = lp["Wih_f"].T
  whhf = lp["Whh_f"].T
  bf = (lp["bih_f"] + lp["bhh_f"])[None, :]
  wihb = lp["Wih_b"].T
  whhb = lp["Whh_b"].T
  bb = (lp["bih_b"] + lp["bhh_b"])[None, :]
  att = params["out_jk"]["att_W"]
  wf = jnp.pad(att[:, :128].T, ((0, 0), (0, 7)))
  wb = jnp.pad(att[:, 128:].T, ((0, 0), (0, 7)))

  ones128 = jnp.ones((K, 128), jnp.float32)
  zeros128 = jnp.zeros((ROWS_PER_TILE, 128), jnp.float32)

  deg = _degree(dst, zeros128, ones128)
  dega, degb = deg[0], deg[1]

  u1 = _tc_a(x, w_in, dega, degb)
  agg1 = _scatter128(u1, src, dst, zeros128)
  h1, u2 = _tc_b(agg1[0], agg1[1], u1, dega, degb, b_in, w1)
  agg2 = _scatter128(u2, src, dst, zeros128)
  u3 = _tc_c(agg2[0], agg2[1], u2, h1, dega, degb, b1,
             wihf, whhf, bf, wihb, whhb, bb, wf, wb, w_out)
  agg3 = _scatter128(u3, src, dst, zeros128)
  return _tc_d(agg3[0], agg3[1], u3, dega, degb, b_out)


# speedup vs baseline: 15.0432x; 1.0202x over previous
import functools

import jax
import jax.numpy as jnp
from jax import lax
from jax.experimental import pallas as pl
from jax.experimental.pallas import tpu as pltpu
from jax.experimental.pallas import tpu_sc as plsc

N = 10000
NP = 10240
E = 320000
NC = 2
NS = 16
NW = NC * NS
EW = E // NW
K = 80
ROWS_PER_TILE = NP // NS


def _make_scatter(W):
  mesh = plsc.VectorSubcoreMesh(core_axis_name="c", subcore_axis_name="s")

  @functools.partial(
      pl.kernel,
      out_type=jax.ShapeDtypeStruct((NC, NP, W), jnp.float32),
      mesh=mesh,
      scratch_types=[
          pltpu.VMEM((K,), jnp.int32),
          pltpu.VMEM((K,), jnp.int32),
          pltpu.VMEM((K, W), jnp.float32),
          pltpu.VMEM((K,), jnp.int32),
          pltpu.VMEM((K,), jnp.int32),
          pltpu.VMEM((K, W), jnp.float32),
          pltpu.VMEM_SHARED((NP, W), jnp.float32),
          pltpu.SemaphoreType.DMA,
          pltpu.SemaphoreType.DMA,
          pltpu.SemaphoreType.DMA,
          pltpu.SemaphoreType.DMA,
      ],
  )
  def scatter_kernel(u_hbm, src_hbm, dst_hbm, zeros_hbm, out_hbm,
                     sidx_a, didx_a, rows_a, sidx_b, didx_b, rows_b,
                     acc, sem_a, sem_b, sem_sa, sem_sb):
    c = lax.axis_index("c")
    s = lax.axis_index("s")
    stripe = pl.ds(s * ROWS_PER_TILE, ROWS_PER_TILE)
    pltpu.sync_copy(zeros_hbm, acc.at[stripe])
    plsc.subcore_barrier()

    base0 = (c * NS + s) * EW
    n_iters = EW // K

    def load_idx(base, sidx, didx):
      pltpu.sync_copy(src_hbm.at[pl.ds(base, K)], sidx)
      pltpu.sync_copy(dst_hbm.at[pl.ds(base, K)], didx)

    load_idx(base0, sidx_a, didx_a)
    pltpu.async_copy(u_hbm.at[sidx_a], rows_a, sem_a)
    load_idx(base0 + K, sidx_b, didx_b)
    pltpu.async_copy(u_hbm.at[sidx_b], rows_b, sem_b)

    def body(j, carry):
      pltpu.make_async_copy(u_hbm.at[sidx_a], rows_a, sem_a).wait()
      pltpu.async_copy(rows_a, acc.at[didx_a], sem_sa, add=True)

      pltpu.make_async_copy(u_hbm.at[sidx_b], rows_b, sem_b).wait()
      pltpu.async_copy(rows_b, acc.at[didx_b], sem_sb, add=True)

      @pl.when(j < 61)
      def _():
        pltpu.make_async_copy(rows_a, acc.at[didx_a], sem_sa).wait()
        load_idx(base0 + (2 * j + 2) * K, sidx_a, didx_a)
        pltpu.async_copy(u_hbm.at[sidx_a], rows_a, sem_a)
        pltpu.make_async_copy(rows_b, acc.at[didx_b], sem_sb).wait()
        load_idx(base0 + (2 * j + 3) * K, sidx_b, didx_b)
        pltpu.async_copy(u_hbm.at[sidx_b], rows_b, sem_b)

      return carry

    lax.fori_loop(0, 62, body, 0)
    pltpu.make_async_copy(rows_a, acc.at[didx_a], sem_sa).wait()
    load_idx(base0 + (n_iters - 1) * K, sidx_a, didx_a)
    pltpu.async_copy(u_hbm.at[sidx_a], rows_a, sem_a).wait()
    pltpu.async_copy(rows_a, acc.at[didx_a], sem_sa, add=True)
    pltpu.make_async_copy(rows_a, acc.at[didx_a], sem_sa).wait()
    pltpu.make_async_copy(rows_b, acc.at[didx_b], sem_sb).wait()

    plsc.subcore_barrier()
    pltpu.sync_copy(acc.at[stripe], out_hbm.at[c, stripe])

  return scatter_kernel


def _make_degree():
  mesh = plsc.VectorSubcoreMesh(core_axis_name="c", subcore_axis_name="s")

  @functools.partial(
      pl.kernel,
      out_type=jax.ShapeDtypeStruct((NC, NP, 128), jnp.float32),
      mesh=mesh,
      scratch_types=[
          pltpu.VMEM((K,), jnp.int32),
          pltpu.VMEM((K,), jnp.int32),
          pltpu.VMEM((K, 128), jnp.float32),
          pltpu.VMEM_SHARED((NP, 128), jnp.float32),
          pltpu.SemaphoreType.DMA,
          pltpu.SemaphoreType.DMA,
      ],
  )
  def degree_kernel(dst_hbm, zeros_hbm, ones_hbm, out_hbm,
                    didx_a, didx_b, ones_v, acc, sem_sa, sem_sb):
    c = lax.axis_index("c")
    s = lax.axis_index("s")
    stripe = pl.ds(s * ROWS_PER_TILE, ROWS_PER_TILE)
    pltpu.sync_copy(zeros_hbm, acc.at[stripe])
    pltpu.sync_copy(ones_hbm, ones_v)
    plsc.subcore_barrier()

    base0 = (c * NS + s) * EW
    n_iters = EW // K

    pltpu.sync_copy(dst_hbm.at[pl.ds(base0, K)], didx_a)
    pltpu.async_copy(ones_v, acc.at[didx_a], sem_sa, add=True)
    pltpu.sync_copy(dst_hbm.at[pl.ds(base0 + K, K)], didx_b)
    pltpu.async_copy(ones_v, acc.at[didx_b], sem_sb, add=True)

    def body(j, carry):
      pltpu.make_async_copy(ones_v, acc.at[didx_a], sem_sa).wait()
      pltpu.sync_copy(dst_hbm.at[pl.ds(base0 + (2 * j + 2) * K, K)], didx_a)
      pltpu.async_copy(ones_v, acc.at[didx_a], sem_sa, add=True)
      pltpu.make_async_copy(ones_v, acc.at[didx_b], sem_sb).wait()
      pltpu.sync_copy(dst_hbm.at[pl.ds(base0 + (2 * j + 3) * K, K)], didx_b)
      pltpu.async_copy(ones_v, acc.at[didx_b], sem_sb, add=True)
      return carry

    lax.fori_loop(0, 61, body, 0)
    pltpu.make_async_copy(ones_v, acc.at[didx_a], sem_sa).wait()
    pltpu.sync_copy(dst_hbm.at[pl.ds(base0 + (n_iters - 1) * K, K)], didx_a)
    pltpu.async_copy(ones_v, acc.at[didx_a], sem_sa, add=True)
    pltpu.make_async_copy(ones_v, acc.at[didx_a], sem_sa).wait()
    pltpu.make_async_copy(ones_v, acc.at[didx_b], sem_sb).wait()
    plsc.subcore_barrier()
    pltpu.sync_copy(acc.at[stripe], out_hbm.at[c, stripe])

  return degree_kernel


_scatter128 = _make_scatter(128)
_degree = _make_degree()


def _dis(da_ref, db_ref):
  deg = da_ref[:, 0:1] + db_ref[:, 0:1] + 1.0
  return lax.rsqrt(deg)


def _tc_a(x, w_in, dega, degb, R=1000):
  def body(x_ref, w_ref, da_ref, db_ref, u1_ref):
    dis = _dis(da_ref, db_ref)
    h = jnp.dot(x_ref[...], w_ref[...], preferred_element_type=jnp.float32)
    u1_ref[...] = h * dis

  return pl.pallas_call(
      body,
      grid=(N // R,),
      in_specs=[
          pl.BlockSpec((R, 128), lambda i: (i, 0)),
          pl.BlockSpec((128, 128), lambda i: (0, 0)),
          pl.BlockSpec((R, 128), lambda i: (i, 0)),
          pl.BlockSpec((R, 128), lambda i: (i, 0)),
      ],
      out_specs=pl.BlockSpec((R, 128), lambda i: (i, 0)),
      out_shape=jax.ShapeDtypeStruct((N, 128), jnp.float32),
  )(x, w_in, dega, degb)


def _tc_b(agg1a, agg1b, u1, dega, degb, b_in, w1, R=1000):
  def body(aa_ref, ab_ref, u1_ref, da_ref, db_ref, bin_ref, w1_ref,
           h1_ref, u2_ref):
    dis = _dis(da_ref, db_ref)
    h1 = dis * (aa_ref[...] + ab_ref[...] + u1_ref[...]) + bin_ref[...]
    h1 = jnp.maximum(h1, 0.0)
    h1_ref[...] = h1
    u2_ref[...] = jnp.dot(h1, w1_ref[...],
                          preferred_element_type=jnp.float32) * dis

  return pl.pallas_call(
      body,
      grid=(N // R,),
      in_specs=[
          pl.BlockSpec((R, 128), lambda i: (i, 0)),
          pl.BlockSpec((R, 128), lambda i: (i, 0)),
          pl.BlockSpec((R, 128), lambda i: (i, 0)),
          pl.BlockSpec((R, 128), lambda i: (i, 0)),
          pl.BlockSpec((R, 128), lambda i: (i, 0)),
          pl.BlockSpec((1, 128), lambda i: (0, 0)),
          pl.BlockSpec((128, 128), lambda i: (0, 0)),
      ],
      out_specs=[
          pl.BlockSpec((R, 128), lambda i: (i, 0)),
          pl.BlockSpec((R, 128), lambda i: (i, 0)),
      ],
      out_shape=[
          jax.ShapeDtypeStruct((N, 128), jnp.float32),
          jax.ShapeDtypeStruct((N, 128), jnp.float32),
      ],
  )(agg1a, agg1b, u1, dega, degb, b_in, w1)


def _lstm_cell(g, c_prev):
  i = jax.nn.sigmoid(g[:, 0:128])
  f = jax.nn.sigmoid(g[:, 128:256])
  gg = jnp.tanh(g[:, 256:384])
  o = jax.nn.sigmoid(g[:, 384:512])
  c = f * c_prev + i * gg
  return o * jnp.tanh(c), c


def _tc_c(agg2a, agg2b, u2, h1, dega, degb, b1,
          wihf, whhf, bf, wihb, whhb, bb, wf, wb, w_out, R=400):
  def body(aa_ref, ab_ref, u2_ref, h1_ref, da_ref, db_ref, b1_ref,
           wihf_ref, whhf_ref, bf_ref, wihb_ref, whhb_ref, bb_ref,
           wf_ref, wb_ref, wo_ref, u3_ref):
    dis = _dis(da_ref, db_ref)
    h2 = dis * (aa_ref[...] + ab_ref[...] + u2_ref[...]) + b1_ref[...]
    h2 = jnp.maximum(h2, 0.0)
    h1 = h1_ref[...]

    def mm(a, b):
      return jnp.dot(a, b, preferred_element_type=jnp.float32)

    hf0, cf0 = _lstm_cell(mm(h1, wihf_ref[...]) + bf_ref[...], 0.0)
    hf1, _ = _lstm_cell(mm(h2, wihf_ref[...]) + mm(hf0, whhf_ref[...])
                        + bf_ref[...], cf0)
    hb0, cb0 = _lstm_cell(mm(h2, wihb_ref[...]) + bb_ref[...], 0.0)
    hb1, _ = _lstm_cell(mm(h1, wihb_ref[...]) + mm(hb0, whhb_ref[...])
                        + bb_ref[...], cb0)
    a0 = mm(hf0, wf_ref[...]) + mm(hb1, wb_ref[...])
    a1 = mm(hf1, wf_ref[...]) + mm(hb0, wb_ref[...])
    alpha = jax.nn.sigmoid(a0[:, 0:1] - a1[:, 0:1])
    h3 = alpha * h1 + (1.0 - alpha) * h2
    u3_ref[...] = mm(h3, wo_ref[...]) * dis

  return pl.pallas_call(
      body,
      grid=(N // R,),
      in_specs=[
          pl.BlockSpec((R, 128), lambda i: (i, 0)),
          pl.BlockSpec((R, 128), lambda i: (i, 0)),
          pl.BlockSpec((R, 128), lambda i: (i, 0)),
          pl.BlockSpec((R, 128), lambda i: (i, 0)),
          pl.BlockSpec((R, 128), lambda i: (i, 0)),
          pl.BlockSpec((R, 128), lambda i: (i, 0)),
          pl.BlockSpec((1, 128), lambda i: (0, 0)),
          pl.BlockSpec((128, 512), lambda i: (0, 0)),
          pl.BlockSpec((128, 512), lambda i: (0, 0)),
          pl.BlockSpec((1, 512), lambda i: (0, 0)),
          pl.BlockSpec((128, 512), lambda i: (0, 0)),
          pl.BlockSpec((128, 512), lambda i: (0, 0)),
          pl.BlockSpec((1, 512), lambda i: (0, 0)),
          pl.BlockSpec((128, 8), lambda i: (0, 0)),
          pl.BlockSpec((128, 8), lambda i: (0, 0)),
          pl.BlockSpec((128, 128), lambda i: (0, 0)),
      ],
      out_specs=pl.BlockSpec((R, 128), lambda i: (i, 0)),
      out_shape=jax.ShapeDtypeStruct((N, 128), jnp.float32),
  )(agg2a, agg2b, u2, h1, dega, degb, b1,
    wihf, whhf, bf, wihb, whhb, bb, wf, wb, w_out)


def _tc_d(agg3a, agg3b, u3, dega, degb, b_out, R=1000):
  def body(aa_ref, ab_ref, u3_ref, da_ref, db_ref, bo_ref, out_ref):
    dis = _dis(da_ref, db_ref)
    full = dis * (aa_ref[...] + ab_ref[...] + u3_ref[...])
    out_ref[...] = full[:, 0:40] + bo_ref[...]

  return pl.pallas_call(
      body,
      grid=(N // R,),
      in_specs=[
          pl.BlockSpec((R, 128), lambda i: (i, 0)),
          pl.BlockSpec((R, 128), lambda i: (i, 0)),
          pl.BlockSpec((R, 128), lambda i: (i, 0)),
          pl.BlockSpec((R, 128), lambda i: (i, 0)),
          pl.BlockSpec((R, 128), lambda i: (i, 0)),
          pl.BlockSpec((1, 40), lambda i: (0, 0)),
      ],
      out_specs=pl.BlockSpec((R, 40), lambda i: (i, 0)),
      out_shape=jax.ShapeDtypeStruct((N, 40), jnp.float32),
  )(agg3a, agg3b, u3, dega, degb, b_out)


@jax.jit
def kernel(x, edge_index, params):
  src = edge_index[0]
  dst = edge_index[1]

  w_in = params["in_gc"]["W"]
  b_in = params["in_gc"]["b"][None, :]
  w1 = params["gc1"]["W"]
  b1 = params["gc1"]["b"][None, :]
  w_out = jnp.pad(params["out_gc"]["W"], ((0, 0), (0, 88)))
  b_out = params["out_gc"]["b"][None, :]
  lp = params["out_jk"]["lstm"]
  wihf = lp["Wih_f"].T
  whhf = lp["Whh_f"].T
  bf = (lp["bih_f"] + lp["bhh_f"])[None, :]
  wihb = lp["Wih_b"].T
  whhb = lp["Whh_b"].T
  bb = (lp["bih_b"] + lp["bhh_b"])[None, :]
  att = params["out_jk"]["att_W"]
  wf = jnp.pad(att[:, :128].T, ((0, 0), (0, 7)))
  wb = jnp.pad(att[:, 128:].T, ((0, 0), (0, 7)))

  ones128 = jnp.ones((K, 128), jnp.float32)
  zeros128 = jnp.zeros((ROWS_PER_TILE, 128), jnp.float32)

  deg = _degree(dst, zeros128, ones128)
  dega, degb = deg[0], deg[1]

  u1 = _tc_a(x, w_in, dega, degb)
  agg1 = _scatter128(u1, src, dst, zeros128)
  h1, u2 = _tc_b(agg1[0], agg1[1], u1, dega, degb, b_in, w1)
  agg2 = _scatter128(u2, src, dst, zeros128)
  u3 = _tc_c(agg2[0], agg2[1], u2, h1, dega, degb, b1,
             wihf, whhf, bf, wihb, whhb, bb, wf, wb, w_out)
  agg3 = _scatter128(u3, src, dst, zeros128)
  return _tc_d(agg3[0], agg3[1], u3, dega, degb, b_out)


# speedup vs baseline: 17.5648x; 1.1676x over previous
import functools

import jax
import jax.numpy as jnp
from jax import lax
from jax.experimental import pallas as pl
from jax.experimental.pallas import tpu as pltpu
from jax.experimental.pallas import tpu_sc as plsc

N = 10000
NP = 10240
E = 320000
NC = 2
NS = 16
NW = NC * NS
EW = E // NW
K = 80
ROWS_PER_TILE = NP // NS


def _make_scatter(W):
  mesh = plsc.VectorSubcoreMesh(core_axis_name="c", subcore_axis_name="s")

  @functools.partial(
      pl.kernel,
      out_type=jax.ShapeDtypeStruct((NC, NP, W), jnp.float32),
      mesh=mesh,
      scratch_types=(
          [pltpu.VMEM((K,), jnp.int32)] * 4
          + [pltpu.VMEM((K,), jnp.int32)] * 4
          + [pltpu.VMEM((K, W), jnp.float32)] * 4
          + [pltpu.VMEM_SHARED((NP, W), jnp.float32)]
          + [pltpu.SemaphoreType.DMA] * 12
      ),
  )
  def scatter_kernel(u_hbm, src_hbm, dst_hbm, zeros_hbm, out_hbm, *scr):
    sidx = scr[0:4]
    didx = scr[4:8]
    rows = scr[8:12]
    acc = scr[12]
    sem_g = scr[13:17]
    sem_s = scr[17:21]
    sem_i = scr[21:25]
    c = lax.axis_index("c")
    s = lax.axis_index("s")
    stripe = pl.ds(s * ROWS_PER_TILE, ROWS_PER_TILE)
    pltpu.sync_copy(zeros_hbm, acc.at[stripe])
    plsc.subcore_barrier()

    base0 = (c * NS + s) * EW

    def issue_idx(i, q):
      pltpu.async_copy(src_hbm.at[pl.ds(base0 + i * K, K)], sidx[q], sem_i[q])
      pltpu.async_copy(dst_hbm.at[pl.ds(base0 + i * K, K)], didx[q], sem_i[q])

    def wait_idx(q):
      pltpu.make_async_copy(src_hbm.at[pl.ds(base0, K)], sidx[q],
                            sem_i[q]).wait()
      pltpu.make_async_copy(dst_hbm.at[pl.ds(base0, K)], didx[q],
                            sem_i[q]).wait()

    def issue_gather(q):
      pltpu.async_copy(u_hbm.at[sidx[q]], rows[q], sem_g[q])

    def wait_gather(q):
      pltpu.make_async_copy(u_hbm.at[sidx[q]], rows[q], sem_g[q]).wait()

    def issue_scatter(q):
      pltpu.async_copy(rows[q], acc.at[didx[q]], sem_s[q], add=True)

    def wait_scatter(q):
      pltpu.make_async_copy(rows[q], acc.at[didx[q]], sem_s[q]).wait()

    pltpu.sync_copy(src_hbm.at[pl.ds(base0, K)], sidx[0])
    pltpu.sync_copy(dst_hbm.at[pl.ds(base0, K)], didx[0])
    issue_gather(0)
    issue_idx(1, 1)

    def body(j, carry):
      for k in range(4):
        q = k
        q2 = (k + 2) % 4
        q3 = (k + 1) % 4
        wait_gather(q)
        issue_scatter(q)
        if k < 2:
          @pl.when(j > 0)
          def _(q2=q2):
            wait_scatter(q2)
          issue_idx(4 * j + k + 2, q2)
        elif k == 2:
          wait_scatter(q2)
          issue_idx(4 * j + k + 2, q2)
        else:
          wait_scatter(q2)
          @pl.when(j < 30)
          def _(q2=q2):
            issue_idx(4 * j + k + 2, q2)
        wait_idx(q3)
        issue_gather(q3)
      return carry

    lax.fori_loop(0, 31, body, 0)
    wait_gather(0)
    issue_scatter(0)
    wait_scatter(0)
    wait_scatter(2)
    wait_scatter(3)

    plsc.subcore_barrier()
    pltpu.sync_copy(acc.at[stripe], out_hbm.at[c, stripe])

  return scatter_kernel


def _make_degree():
  mesh = plsc.VectorSubcoreMesh(core_axis_name="c", subcore_axis_name="s")

  @functools.partial(
      pl.kernel,
      out_type=jax.ShapeDtypeStruct((NC, NP, 128), jnp.float32),
      mesh=mesh,
      scratch_types=[
          pltpu.VMEM((K,), jnp.int32),
          pltpu.VMEM((K,), jnp.int32),
          pltpu.VMEM((K, 128), jnp.float32),
          pltpu.VMEM_SHARED((NP, 128), jnp.float32),
          pltpu.SemaphoreType.DMA,
          pltpu.SemaphoreType.DMA,
      ],
  )
  def degree_kernel(dst_hbm, zeros_hbm, ones_hbm, out_hbm,
                    didx_a, didx_b, ones_v, acc, sem_sa, sem_sb):
    c = lax.axis_index("c")
    s = lax.axis_index("s")
    stripe = pl.ds(s * ROWS_PER_TILE, ROWS_PER_TILE)
    pltpu.sync_copy(zeros_hbm, acc.at[stripe])
    pltpu.sync_copy(ones_hbm, ones_v)
    plsc.subcore_barrier()

    base0 = (c * NS + s) * EW
    n_iters = EW // K

    pltpu.sync_copy(dst_hbm.at[pl.ds(base0, K)], didx_a)
    pltpu.async_copy(ones_v, acc.at[didx_a], sem_sa, add=True)
    pltpu.sync_copy(dst_hbm.at[pl.ds(base0 + K, K)], didx_b)
    pltpu.async_copy(ones_v, acc.at[didx_b], sem_sb, add=True)

    def body(j, carry):
      pltpu.make_async_copy(ones_v, acc.at[didx_a], sem_sa).wait()
      pltpu.sync_copy(dst_hbm.at[pl.ds(base0 + (2 * j + 2) * K, K)], didx_a)
      pltpu.async_copy(ones_v, acc.at[didx_a], sem_sa, add=True)
      pltpu.make_async_copy(ones_v, acc.at[didx_b], sem_sb).wait()
      pltpu.sync_copy(dst_hbm.at[pl.ds(base0 + (2 * j + 3) * K, K)], didx_b)
      pltpu.async_copy(ones_v, acc.at[didx_b], sem_sb, add=True)
      return carry

    lax.fori_loop(0, 61, body, 0)
    pltpu.make_async_copy(ones_v, acc.at[didx_a], sem_sa).wait()
    pltpu.sync_copy(dst_hbm.at[pl.ds(base0 + (n_iters - 1) * K, K)], didx_a)
    pltpu.async_copy(ones_v, acc.at[didx_a], sem_sa, add=True)
    pltpu.make_async_copy(ones_v, acc.at[didx_a], sem_sa).wait()
    pltpu.make_async_copy(ones_v, acc.at[didx_b], sem_sb).wait()
    plsc.subcore_barrier()
    pltpu.sync_copy(acc.at[stripe], out_hbm.at[c, stripe])

  return degree_kernel


_scatter128 = _make_scatter(128)
_degree = _make_degree()


def _dis(da_ref, db_ref):
  deg = da_ref[:, 0:1] + db_ref[:, 0:1] + 1.0
  return lax.rsqrt(deg)


def _tc_a(x, w_in, dega, degb, R=1000):
  def body(x_ref, w_ref, da_ref, db_ref, u1_ref):
    dis = _dis(da_ref, db_ref)
    h = jnp.dot(x_ref[...], w_ref[...], preferred_element_type=jnp.float32)
    u1_ref[...] = h * dis

  return pl.pallas_call(
      body,
      grid=(N // R,),
      in_specs=[
          pl.BlockSpec((R, 128), lambda i: (i, 0)),
          pl.BlockSpec((128, 128), lambda i: (0, 0)),
          pl.BlockSpec((R, 128), lambda i: (i, 0)),
          pl.BlockSpec((R, 128), lambda i: (i, 0)),
      ],
      out_specs=pl.BlockSpec((R, 128), lambda i: (i, 0)),
      out_shape=jax.ShapeDtypeStruct((N, 128), jnp.float32),
  )(x, w_in, dega, degb)


def _tc_b(agg1a, agg1b, u1, dega, degb, b_in, w1, R=1000):
  def body(aa_ref, ab_ref, u1_ref, da_ref, db_ref, bin_ref, w1_ref,
           h1_ref, u2_ref):
    dis = _dis(da_ref, db_ref)
    h1 = dis * (aa_ref[...] + ab_ref[...] + u1_ref[...]) + bin_ref[...]
    h1 = jnp.maximum(h1, 0.0)
    h1_ref[...] = h1
    u2_ref[...] = jnp.dot(h1, w1_ref[...],
                          preferred_element_type=jnp.float32) * dis

  return pl.pallas_call(
      body,
      grid=(N // R,),
      in_specs=[
          pl.BlockSpec((R, 128), lambda i: (i, 0)),
          pl.BlockSpec((R, 128), lambda i: (i, 0)),
          pl.BlockSpec((R, 128), lambda i: (i, 0)),
          pl.BlockSpec((R, 128), lambda i: (i, 0)),
          pl.BlockSpec((R, 128), lambda i: (i, 0)),
          pl.BlockSpec((1, 128), lambda i: (0, 0)),
          pl.BlockSpec((128, 128), lambda i: (0, 0)),
      ],
      out_specs=[
          pl.BlockSpec((R, 128), lambda i: (i, 0)),
          pl.BlockSpec((R, 128), lambda i: (i, 0)),
      ],
      out_shape=[
          jax.ShapeDtypeStruct((N, 128), jnp.float32),
          jax.ShapeDtypeStruct((N, 128), jnp.float32),
      ],
  )(agg1a, agg1b, u1, dega, degb, b_in, w1)


def _lstm_cell(g, c_prev):
  i = jax.nn.sigmoid(g[:, 0:128])
  f = jax.nn.sigmoid(g[:, 128:256])
  gg = jnp.tanh(g[:, 256:384])
  o = jax.nn.sigmoid(g[:, 384:512])
  c = f * c_prev + i * gg
  return o * jnp.tanh(c), c


def _tc_c(agg2a, agg2b, u2, h1, dega, degb, b1,
          wihf, whhf, bf, wihb, whhb, bb, wf, wb, w_out, R=400):
  def body(aa_ref, ab_ref, u2_ref, h1_ref, da_ref, db_ref, b1_ref,
           wihf_ref, whhf_ref, bf_ref, wihb_ref, whhb_ref, bb_ref,
           wf_ref, wb_ref, wo_ref, u3_ref):
    dis = _dis(da_ref, db_ref)
    h2 = dis * (aa_ref[...] + ab_ref[...] + u2_ref[...]) + b1_ref[...]
    h2 = jnp.maximum(h2, 0.0)
    h1 = h1_ref[...]

    def mm(a, b):
      return jnp.dot(a, b, preferred_element_type=jnp.float32)

    hf0, cf0 = _lstm_cell(mm(h1, wihf_ref[...]) + bf_ref[...], 0.0)
    hf1, _ = _lstm_cell(mm(h2, wihf_ref[...]) + mm(hf0, whhf_ref[...])
                        + bf_ref[...], cf0)
    hb0, cb0 = _lstm_cell(mm(h2, wihb_ref[...]) + bb_ref[...], 0.0)
    hb1, _ = _lstm_cell(mm(h1, wihb_ref[...]) + mm(hb0, whhb_ref[...])
                        + bb_ref[...], cb0)
    a0 = mm(hf0, wf_ref[...]) + mm(hb1, wb_ref[...])
    a1 = mm(hf1, wf_ref[...]) + mm(hb0, wb_ref[...])
    alpha = jax.nn.sigmoid(a0[:, 0:1] - a1[:, 0:1])
    h3 = alpha * h1 + (1.0 - alpha) * h2
    u3_ref[...] = mm(h3, wo_ref[...]) * dis

  return pl.pallas_call(
      body,
      grid=(N // R,),
      in_specs=[
          pl.BlockSpec((R, 128), lambda i: (i, 0)),
          pl.BlockSpec((R, 128), lambda i: (i, 0)),
          pl.BlockSpec((R, 128), lambda i: (i, 0)),
          pl.BlockSpec((R, 128), lambda i: (i, 0)),
          pl.BlockSpec((R, 128), lambda i: (i, 0)),
          pl.BlockSpec((R, 128), lambda i: (i, 0)),
          pl.BlockSpec((1, 128), lambda i: (0, 0)),
          pl.BlockSpec((128, 512), lambda i: (0, 0)),
          pl.BlockSpec((128, 512), lambda i: (0, 0)),
          pl.BlockSpec((1, 512), lambda i: (0, 0)),
          pl.BlockSpec((128, 512), lambda i: (0, 0)),
          pl.BlockSpec((128, 512), lambda i: (0, 0)),
          pl.BlockSpec((1, 512), lambda i: (0, 0)),
          pl.BlockSpec((128, 8), lambda i: (0, 0)),
          pl.BlockSpec((128, 8), lambda i: (0, 0)),
          pl.BlockSpec((128, 128), lambda i: (0, 0)),
      ],
      out_specs=pl.BlockSpec((R, 128), lambda i: (i, 0)),
      out_shape=jax.ShapeDtypeStruct((N, 128), jnp.float32),
  )(agg2a, agg2b, u2, h1, dega, degb, b1,
    wihf, whhf, bf, wihb, whhb, bb, wf, wb, w_out)


def _tc_d(agg3a, agg3b, u3, dega, degb, b_out, R=1000):
  def body(aa_ref, ab_ref, u3_ref, da_ref, db_ref, bo_ref, out_ref):
    dis = _dis(da_ref, db_ref)
    full = dis * (aa_ref[...] + ab_ref[...] + u3_ref[...])
    out_ref[...] = full[:, 0:40] + bo_ref[...]

  return pl.pallas_call(
      body,
      grid=(N // R,),
      in_specs=[
          pl.BlockSpec((R, 128), lambda i: (i, 0)),
          pl.BlockSpec((R, 128), lambda i: (i, 0)),
          pl.BlockSpec((R, 128), lambda i: (i, 0)),
          pl.BlockSpec((R, 128), lambda i: (i, 0)),
          pl.BlockSpec((R, 128), lambda i: (i, 0)),
          pl.BlockSpec((1, 40), lambda i: (0, 0)),
      ],
      out_specs=pl.BlockSpec((R, 40), lambda i: (i, 0)),
      out_shape=jax.ShapeDtypeStruct((N, 40), jnp.float32),
  )(agg3a, agg3b, u3, dega, degb, b_out)


@jax.jit
def kernel(x, edge_index, params):
  src = edge_index[0]
  dst = edge_index[1]

  w_in = params["in_gc"]["W"]
  b_in = params["in_gc"]["b"][None, :]
  w1 = params["gc1"]["W"]
  b1 = params["gc1"]["b"][None, :]
  w_out = jnp.pad(params["out_gc"]["W"], ((0, 0), (0, 88)))
  b_out = params["out_gc"]["b"][None, :]
  lp = params["out_jk"]["lstm"]
  wihf = lp["Wih_f"].T
  whhf = lp["Whh_f"].T
  bf = (lp["bih_f"] + lp["bhh_f"])[None, :]
  wihb = lp["Wih_b"].T
  whhb = lp["Whh_b"].T
  bb = (lp["bih_b"] + lp["bhh_b"])[None, :]
  att = params["out_jk"]["att_W"]
  wf = jnp.pad(att[:, :128].T, ((0, 0), (0, 7)))
  wb = jnp.pad(att[:, 128:].T, ((0, 0), (0, 7)))

  ones128 = jnp.ones((K, 128), jnp.float32)
  zeros128 = jnp.zeros((ROWS_PER_TILE, 128), jnp.float32)

  deg = _degree(dst, zeros128, ones128)
  dega, degb = deg[0], deg[1]

  u1 = _tc_a(x, w_in, dega, degb)
  agg1 = _scatter128(u1, src, dst, zeros128)
  h1, u2 = _tc_b(agg1[0], agg1[1], u1, dega, degb, b_in, w1)
  agg2 = _scatter128(u2, src, dst, zeros128)
  u3 = _tc_c(agg2[0], agg2[1], u2, h1, dega, degb, b1,
             wihf, whhf, bf, wihb, whhb, bb, wf, wb, w_out)
  agg3 = _scatter128(u3, src, dst, zeros128)
  return _tc_d(agg3[0], agg3[1], u3, dega, degb, b_out)


# speedup vs baseline: 18.2923x; 1.0414x over previous
import functools

import jax
import jax.numpy as jnp
from jax import lax
from jax.experimental import pallas as pl
from jax.experimental.pallas import tpu as pltpu
from jax.experimental.pallas import tpu_sc as plsc

N = 10000
NP = 10240
E = 320000
NC = 2
NS = 16
NW = NC * NS
EW = E // NW
K = 80
ROWS_PER_TILE = NP // NS


def _make_scatter(W):
  mesh = plsc.VectorSubcoreMesh(core_axis_name="c", subcore_axis_name="s")

  @functools.partial(
      pl.kernel,
      out_type=jax.ShapeDtypeStruct((NC, NP, W), jnp.float32),
      mesh=mesh,
      scratch_types=(
          [pltpu.VMEM((K,), jnp.int32)] * 4
          + [pltpu.VMEM((K,), jnp.int32)] * 4
          + [pltpu.VMEM((K, W), jnp.float32)] * 4
          + [pltpu.VMEM_SHARED((NP, W), jnp.float32)]
          + [pltpu.SemaphoreType.DMA] * 12
      ),
  )
  def scatter_kernel(u_hbm, src_hbm, dst_hbm, zeros_hbm, out_hbm, *scr):
    sidx = scr[0:4]
    didx = scr[4:8]
    rows = scr[8:12]
    acc = scr[12]
    sem_g = scr[13:17]
    sem_s = scr[17:21]
    sem_i = scr[21:25]
    c = lax.axis_index("c")
    s = lax.axis_index("s")
    stripe = pl.ds(s * ROWS_PER_TILE, ROWS_PER_TILE)
    pltpu.sync_copy(zeros_hbm, acc.at[stripe])
    plsc.subcore_barrier()

    base0 = (c * NS + s) * EW

    def issue_idx(i, q):
      pltpu.async_copy(src_hbm.at[pl.ds(base0 + i * K, K)], sidx[q], sem_i[q])
      pltpu.async_copy(dst_hbm.at[pl.ds(base0 + i * K, K)], didx[q], sem_i[q])

    def wait_idx(q):
      pltpu.make_async_copy(src_hbm.at[pl.ds(base0, K)], sidx[q],
                            sem_i[q]).wait()
      pltpu.make_async_copy(dst_hbm.at[pl.ds(base0, K)], didx[q],
                            sem_i[q]).wait()

    def issue_gather(q):
      pltpu.async_copy(u_hbm.at[sidx[q]], rows[q], sem_g[q])

    def wait_gather(q):
      pltpu.make_async_copy(u_hbm.at[sidx[q]], rows[q], sem_g[q]).wait()

    def issue_scatter(q):
      pltpu.async_copy(rows[q], acc.at[didx[q]], sem_s[q], add=True)

    def wait_scatter(q):
      pltpu.make_async_copy(rows[q], acc.at[didx[q]], sem_s[q]).wait()

    pltpu.sync_copy(src_hbm.at[pl.ds(base0, K)], sidx[0])
    pltpu.sync_copy(dst_hbm.at[pl.ds(base0, K)], didx[0])
    issue_gather(0)
    issue_idx(1, 1)

    def body(j, carry):
      for k in range(4):
        q = k
        q2 = (k + 2) % 4
        q3 = (k + 1) % 4
        wait_gather(q)
        issue_scatter(q)
        if k < 2:
          @pl.when(j > 0)
          def _(q2=q2):
            wait_scatter(q2)
          issue_idx(4 * j + k + 2, q2)
        elif k == 2:
          wait_scatter(q2)
          issue_idx(4 * j + k + 2, q2)
        else:
          wait_scatter(q2)
          @pl.when(j < 30)
          def _(q2=q2):
            issue_idx(4 * j + k + 2, q2)
        wait_idx(q3)
        issue_gather(q3)
      return carry

    lax.fori_loop(0, 31, body, 0)
    wait_gather(0)
    issue_scatter(0)
    wait_scatter(0)
    wait_scatter(2)
    wait_scatter(3)

    plsc.subcore_barrier()
    pltpu.sync_copy(acc.at[stripe], out_hbm.at[c, stripe])

  return scatter_kernel


def _make_degree():
  mesh = plsc.VectorSubcoreMesh(core_axis_name="c", subcore_axis_name="s")

  @functools.partial(
      pl.kernel,
      out_type=jax.ShapeDtypeStruct((NC, NP, 128), jnp.float32),
      mesh=mesh,
      scratch_types=[
          pltpu.VMEM((K,), jnp.int32),
          pltpu.VMEM((K,), jnp.int32),
          pltpu.VMEM((K, 128), jnp.float32),
          pltpu.VMEM_SHARED((NP, 128), jnp.float32),
          pltpu.SemaphoreType.DMA,
          pltpu.SemaphoreType.DMA,
      ],
  )
  def degree_kernel(dst_hbm, zeros_hbm, ones_hbm, out_hbm,
                    didx_a, didx_b, ones_v, acc, sem_sa, sem_sb):
    c = lax.axis_index("c")
    s = lax.axis_index("s")
    stripe = pl.ds(s * ROWS_PER_TILE, ROWS_PER_TILE)
    pltpu.sync_copy(zeros_hbm, acc.at[stripe])
    pltpu.sync_copy(ones_hbm, ones_v)
    plsc.subcore_barrier()

    base0 = (c * NS + s) * EW
    n_iters = EW // K

    pltpu.sync_copy(dst_hbm.at[pl.ds(base0, K)], didx_a)
    pltpu.async_copy(ones_v, acc.at[didx_a], sem_sa, add=True)
    pltpu.sync_copy(dst_hbm.at[pl.ds(base0 + K, K)], didx_b)
    pltpu.async_copy(ones_v, acc.at[didx_b], sem_sb, add=True)

    def body(j, carry):
      pltpu.make_async_copy(ones_v, acc.at[didx_a], sem_sa).wait()
      pltpu.sync_copy(dst_hbm.at[pl.ds(base0 + (2 * j + 2) * K, K)], didx_a)
      pltpu.async_copy(ones_v, acc.at[didx_a], sem_sa, add=True)
      pltpu.make_async_copy(ones_v, acc.at[didx_b], sem_sb).wait()
      pltpu.sync_copy(dst_hbm.at[pl.ds(base0 + (2 * j + 3) * K, K)], didx_b)
      pltpu.async_copy(ones_v, acc.at[didx_b], sem_sb, add=True)
      return carry

    lax.fori_loop(0, 61, body, 0)
    pltpu.make_async_copy(ones_v, acc.at[didx_a], sem_sa).wait()
    pltpu.sync_copy(dst_hbm.at[pl.ds(base0 + (n_iters - 1) * K, K)], didx_a)
    pltpu.async_copy(ones_v, acc.at[didx_a], sem_sa, add=True)
    pltpu.make_async_copy(ones_v, acc.at[didx_a], sem_sa).wait()
    pltpu.make_async_copy(ones_v, acc.at[didx_b], sem_sb).wait()
    plsc.subcore_barrier()
    pltpu.sync_copy(acc.at[stripe], out_hbm.at[c, stripe])

  return degree_kernel


_scatter128 = _make_scatter(128)
_degree = _make_degree()


def _tc_a(x, w_in, deg, R=1000):
  def body(x_ref, w_ref, dg_ref, u1_ref, dis_ref):
    deg = dg_ref[0][:, 0:1] + dg_ref[1][:, 0:1] + 1.0
    dis = lax.rsqrt(deg)
    h = jnp.dot(x_ref[...], w_ref[...], preferred_element_type=jnp.float32)
    u1_ref[...] = h * dis
    dis_ref[...] = jnp.broadcast_to(dis, (R, 8))

  return pl.pallas_call(
      body,
      grid=(N // R,),
      in_specs=[
          pl.BlockSpec((R, 128), lambda i: (i, 0)),
          pl.BlockSpec((128, 128), lambda i: (0, 0)),
          pl.BlockSpec((2, R, 128), lambda i: (0, i, 0)),
      ],
      out_specs=[
          pl.BlockSpec((R, 128), lambda i: (i, 0)),
          pl.BlockSpec((R, 8), lambda i: (i, 0)),
      ],
      out_shape=[
          jax.ShapeDtypeStruct((N, 128), jnp.float32),
          jax.ShapeDtypeStruct((N, 8), jnp.float32),
      ],
  )(x, w_in, deg)


def _tc_b(agg1, u1, dis8, b_in, w1, R=1000):
  def body(ag_ref, u1_ref, dis_ref, bin_ref, w1_ref, h1_ref, u2_ref):
    dis = dis_ref[:, 0:1]
    h1 = dis * (ag_ref[0] + ag_ref[1] + u1_ref[...]) + bin_ref[...]
    h1 = jnp.maximum(h1, 0.0)
    h1_ref[...] = h1
    u2_ref[...] = jnp.dot(h1, w1_ref[...],
                          preferred_element_type=jnp.float32) * dis

  return pl.pallas_call(
      body,
      grid=(N // R,),
      in_specs=[
          pl.BlockSpec((2, R, 128), lambda i: (0, i, 0)),
          pl.BlockSpec((R, 128), lambda i: (i, 0)),
          pl.BlockSpec((R, 8), lambda i: (i, 0)),
          pl.BlockSpec((1, 128), lambda i: (0, 0)),
          pl.BlockSpec((128, 128), lambda i: (0, 0)),
      ],
      out_specs=[
          pl.BlockSpec((R, 128), lambda i: (i, 0)),
          pl.BlockSpec((R, 128), lambda i: (i, 0)),
      ],
      out_shape=[
          jax.ShapeDtypeStruct((N, 128), jnp.float32),
          jax.ShapeDtypeStruct((N, 128), jnp.float32),
      ],
  )(agg1, u1, dis8, b_in, w1)


def _lstm_cell(g, c_prev):
  i = jax.nn.sigmoid(g[:, 0:128])
  f = jax.nn.sigmoid(g[:, 128:256])
  gg = jnp.tanh(g[:, 256:384])
  o = jax.nn.sigmoid(g[:, 384:512])
  c = f * c_prev + i * gg
  return o * jnp.tanh(c), c


def _tc_c(agg2, u2, h1, dis8, b1,
          wihf, whhf, bf, wihb, whhb, bb, wf, wb, w_out, R=400):
  def body(ag_ref, u2_ref, h1_ref, dis_ref, b1_ref,
           wihf_ref, whhf_ref, bf_ref, wihb_ref, whhb_ref, bb_ref,
           wf_ref, wb_ref, wo_ref, u3_ref):
    dis = dis_ref[:, 0:1]
    h2 = dis * (ag_ref[0] + ag_ref[1] + u2_ref[...]) + b1_ref[...]
    h2 = jnp.maximum(h2, 0.0)
    h1 = h1_ref[...]

    def mm(a, b):
      return jnp.dot(a, b, preferred_element_type=jnp.float32)

    def mmb(a, b_ref):
      return jnp.dot(a.astype(jnp.bfloat16),
                     b_ref[...].astype(jnp.bfloat16),
                     preferred_element_type=jnp.float32)

    hf0, cf0 = _lstm_cell(mmb(h1, wihf_ref) + bf_ref[...], 0.0)
    hf1, _ = _lstm_cell(mmb(h2, wihf_ref) + mmb(hf0, whhf_ref)
                        + bf_ref[...], cf0)
    hb0, cb0 = _lstm_cell(mmb(h2, wihb_ref) + bb_ref[...], 0.0)
    hb1, _ = _lstm_cell(mmb(h1, wihb_ref) + mmb(hb0, whhb_ref)
                        + bb_ref[...], cb0)
    a0 = mm(hf0, wf_ref[...]) + mm(hb1, wb_ref[...])
    a1 = mm(hf1, wf_ref[...]) + mm(hb0, wb_ref[...])
    alpha = jax.nn.sigmoid(a0[:, 0:1] - a1[:, 0:1])
    h3 = alpha * h1 + (1.0 - alpha) * h2
    u3_ref[...] = mm(h3, wo_ref[...]) * dis

  return pl.pallas_call(
      body,
      grid=(N // R,),
      in_specs=[
          pl.BlockSpec((2, R, 128), lambda i: (0, i, 0)),
          pl.BlockSpec((R, 128), lambda i: (i, 0)),
          pl.BlockSpec((R, 128), lambda i: (i, 0)),
          pl.BlockSpec((R, 8), lambda i: (i, 0)),
          pl.BlockSpec((1, 128), lambda i: (0, 0)),
          pl.BlockSpec((128, 512), lambda i: (0, 0)),
          pl.BlockSpec((128, 512), lambda i: (0, 0)),
          pl.BlockSpec((1, 512), lambda i: (0, 0)),
          pl.BlockSpec((128, 512), lambda i: (0, 0)),
          pl.BlockSpec((128, 512), lambda i: (0, 0)),
          pl.BlockSpec((1, 512), lambda i: (0, 0)),
          pl.BlockSpec((128, 8), lambda i: (0, 0)),
          pl.BlockSpec((128, 8), lambda i: (0, 0)),
          pl.BlockSpec((128, 128), lambda i: (0, 0)),
      ],
      out_specs=pl.BlockSpec((R, 128), lambda i: (i, 0)),
      out_shape=jax.ShapeDtypeStruct((N, 128), jnp.float32),
  )(agg2, u2, h1, dis8, b1,
    wihf, whhf, bf, wihb, whhb, bb, wf, wb, w_out)


def _tc_d(agg3, u3, dis8, b_out, R=1000):
  def body(ag_ref, u3_ref, dis_ref, bo_ref, out_ref):
    dis = dis_ref[:, 0:1]
    full = dis * (ag_ref[0] + ag_ref[1] + u3_ref[...])
    out_ref[...] = full[:, 0:40] + bo_ref[...]

  return pl.pallas_call(
      body,
      grid=(N // R,),
      in_specs=[
          pl.BlockSpec((2, R, 128), lambda i: (0, i, 0)),
          pl.BlockSpec((R, 128), lambda i: (i, 0)),
          pl.BlockSpec((R, 8), lambda i: (i, 0)),
          pl.BlockSpec((1, 40), lambda i: (0, 0)),
      ],
      out_specs=pl.BlockSpec((R, 40), lambda i: (i, 0)),
      out_shape=jax.ShapeDtypeStruct((N, 40), jnp.float32),
  )(agg3, u3, dis8, b_out)


@jax.jit
def kernel(x, edge_index, params):
  src = edge_index[0]
  dst = edge_index[1]

  w_in = params["in_gc"]["W"]
  b_in = params["in_gc"]["b"][None, :]
  w1 = params["gc1"]["W"]
  b1 = params["gc1"]["b"][None, :]
  w_out = jnp.pad(params["out_gc"]["W"], ((0, 0), (0, 88)))
  b_out = params["out_gc"]["b"][None, :]
  lp = params["out_jk"]["lstm"]
  wihf = lp["Wih_f"].T
  whhf = lp["Whh_f"].T
  bf = (lp["bih_f"] + lp["bhh_f"])[None, :]
  wihb = lp["Wih_b"].T
  whhb = lp["Whh_b"].T
  bb = (lp["bih_b"] + lp["bhh_b"])[None, :]
  att = params["out_jk"]["att_W"]
  wf = jnp.pad(att[:, :128].T, ((0, 0), (0, 7)))
  wb = jnp.pad(att[:, 128:].T, ((0, 0), (0, 7)))

  ones128 = jnp.ones((K, 128), jnp.float32)
  zeros128 = jnp.zeros((ROWS_PER_TILE, 128), jnp.float32)

  deg = _degree(dst, zeros128, ones128)
  u1, dis8 = _tc_a(x, w_in, deg)
  agg1 = _scatter128(u1, src, dst, zeros128)
  h1, u2 = _tc_b(agg1, u1, dis8, b_in, w1)
  agg2 = _scatter128(u2, src, dst, zeros128)
  u3 = _tc_c(agg2, u2, h1, dis8, b1,
             wihf, whhf, bf, wihb, whhb, bb, wf, wb, w_out)
  agg3 = _scatter128(u3, src, dst, zeros128)
  return _tc_d(agg3, u3, dis8, b_out)


# speedup vs baseline: 18.3144x; 1.0012x over previous
import functools

import jax
import jax.numpy as jnp
from jax import lax
from jax.experimental import pallas as pl
from jax.experimental.pallas import tpu as pltpu
from jax.experimental.pallas import tpu_sc as plsc

N = 10000
NP = 10240
E = 320000
NC = 2
NS = 16
NW = NC * NS
EW = E // NW
K = 80
ROWS_PER_TILE = NP // NS


def _make_scatter(W):
  mesh = plsc.VectorSubcoreMesh(core_axis_name="c", subcore_axis_name="s")

  @functools.partial(
      pl.kernel,
      out_type=jax.ShapeDtypeStruct((NC, NP, W), jnp.float32),
      mesh=mesh,
      scratch_types=(
          [pltpu.VMEM((K,), jnp.int32)] * 4
          + [pltpu.VMEM((K,), jnp.int32)] * 4
          + [pltpu.VMEM((K, W), jnp.float32)] * 4
          + [pltpu.VMEM_SHARED((NP, W), jnp.float32)]
          + [pltpu.SemaphoreType.DMA] * 12
      ),
  )
  def scatter_kernel(u_hbm, src_hbm, dst_hbm, zeros_hbm, out_hbm, *scr):
    sidx = scr[0:4]
    didx = scr[4:8]
    rows = scr[8:12]
    acc = scr[12]
    sem_g = scr[13:17]
    sem_s = scr[17:21]
    sem_i = scr[21:25]
    c = lax.axis_index("c")
    s = lax.axis_index("s")
    stripe = pl.ds(s * ROWS_PER_TILE, ROWS_PER_TILE)
    base0 = (c * NS + s) * EW

    def issue_idx(i, q):
      pltpu.async_copy(src_hbm.at[pl.ds(base0 + i * K, K)], sidx[q], sem_i[q])
      pltpu.async_copy(dst_hbm.at[pl.ds(base0 + i * K, K)], didx[q], sem_i[q])

    def wait_idx(q):
      pltpu.make_async_copy(src_hbm.at[pl.ds(base0, K)], sidx[q],
                            sem_i[q]).wait()
      pltpu.make_async_copy(dst_hbm.at[pl.ds(base0, K)], didx[q],
                            sem_i[q]).wait()

    def issue_gather(q):
      pltpu.async_copy(u_hbm.at[sidx[q]], rows[q], sem_g[q])

    def wait_gather(q):
      pltpu.make_async_copy(u_hbm.at[sidx[q]], rows[q], sem_g[q]).wait()

    def issue_scatter(q):
      pltpu.async_copy(rows[q], acc.at[didx[q]], sem_s[q], add=True)

    def wait_scatter(q):
      pltpu.make_async_copy(rows[q], acc.at[didx[q]], sem_s[q]).wait()

    issue_idx(0, 0)
    issue_idx(1, 1)
    pltpu.sync_copy(zeros_hbm, acc.at[stripe])
    wait_idx(0)
    issue_gather(0)
    plsc.subcore_barrier()

    def body(j, carry):
      for k in range(4):
        q = k
        q2 = (k + 2) % 4
        q3 = (k + 1) % 4
        wait_gather(q)
        issue_scatter(q)
        if k < 2:
          @pl.when(j > 0)
          def _(q2=q2):
            wait_scatter(q2)
          issue_idx(4 * j + k + 2, q2)
        elif k == 2:
          wait_scatter(q2)
          issue_idx(4 * j + k + 2, q2)
        else:
          wait_scatter(q2)
          @pl.when(j < 30)
          def _(q2=q2):
            issue_idx(4 * j + k + 2, q2)
        wait_idx(q3)
        issue_gather(q3)
      return carry

    lax.fori_loop(0, 31, body, 0)
    wait_gather(0)
    issue_scatter(0)
    wait_scatter(0)
    wait_scatter(2)
    wait_scatter(3)

    plsc.subcore_barrier()
    pltpu.sync_copy(acc.at[stripe], out_hbm.at[c, stripe])

  return scatter_kernel


def _make_degree():
  mesh = plsc.VectorSubcoreMesh(core_axis_name="c", subcore_axis_name="s")

  @functools.partial(
      pl.kernel,
      out_type=jax.ShapeDtypeStruct((NC, NP, 128), jnp.float32),
      mesh=mesh,
      scratch_types=[
          pltpu.VMEM((K,), jnp.int32),
          pltpu.VMEM((K,), jnp.int32),
          pltpu.VMEM((K, 128), jnp.float32),
          pltpu.VMEM_SHARED((NP, 128), jnp.float32),
          pltpu.SemaphoreType.DMA,
          pltpu.SemaphoreType.DMA,
      ],
  )
  def degree_kernel(dst_hbm, zeros_hbm, ones_hbm, out_hbm,
                    didx_a, didx_b, ones_v, acc, sem_sa, sem_sb):
    c = lax.axis_index("c")
    s = lax.axis_index("s")
    stripe = pl.ds(s * ROWS_PER_TILE, ROWS_PER_TILE)
    base0 = (c * NS + s) * EW
    n_iters = EW // K
    pltpu.sync_copy(dst_hbm.at[pl.ds(base0, K)], didx_a)
    pltpu.sync_copy(dst_hbm.at[pl.ds(base0 + K, K)], didx_b)
    pltpu.sync_copy(ones_hbm, ones_v)
    pltpu.sync_copy(zeros_hbm, acc.at[stripe])
    plsc.subcore_barrier()

    pltpu.async_copy(ones_v, acc.at[didx_a], sem_sa, add=True)
    pltpu.async_copy(ones_v, acc.at[didx_b], sem_sb, add=True)

    def body(j, carry):
      pltpu.make_async_copy(ones_v, acc.at[didx_a], sem_sa).wait()
      pltpu.sync_copy(dst_hbm.at[pl.ds(base0 + (2 * j + 2) * K, K)], didx_a)
      pltpu.async_copy(ones_v, acc.at[didx_a], sem_sa, add=True)
      pltpu.make_async_copy(ones_v, acc.at[didx_b], sem_sb).wait()
      pltpu.sync_copy(dst_hbm.at[pl.ds(base0 + (2 * j + 3) * K, K)], didx_b)
      pltpu.async_copy(ones_v, acc.at[didx_b], sem_sb, add=True)
      return carry

    lax.fori_loop(0, 61, body, 0)
    pltpu.make_async_copy(ones_v, acc.at[didx_a], sem_sa).wait()
    pltpu.sync_copy(dst_hbm.at[pl.ds(base0 + (n_iters - 1) * K, K)], didx_a)
    pltpu.async_copy(ones_v, acc.at[didx_a], sem_sa, add=True)
    pltpu.make_async_copy(ones_v, acc.at[didx_a], sem_sa).wait()
    pltpu.make_async_copy(ones_v, acc.at[didx_b], sem_sb).wait()
    plsc.subcore_barrier()
    pltpu.sync_copy(acc.at[stripe], out_hbm.at[c, stripe])

  return degree_kernel


_scatter128 = _make_scatter(128)
_degree = _make_degree()


def _tc_a0(x, w_in, R=1000):
  def body(x_ref, w_ref, h_ref):
    h_ref[...] = jnp.dot(x_ref[...], w_ref[...],
                         preferred_element_type=jnp.float32)

  return pl.pallas_call(
      body,
      grid=(N // R,),
      in_specs=[
          pl.BlockSpec((R, 128), lambda i: (i, 0)),
          pl.BlockSpec((128, 128), lambda i: (0, 0)),
      ],
      out_specs=pl.BlockSpec((R, 128), lambda i: (i, 0)),
      out_shape=jax.ShapeDtypeStruct((N, 128), jnp.float32),
  )(x, w_in)


def _tc_a1(h_in, deg, R=1000):
  def body(h_ref, dg_ref, u1_ref, dis_ref):
    deg = dg_ref[0][:, 0:1] + dg_ref[1][:, 0:1] + 1.0
    dis = lax.rsqrt(deg)
    u1_ref[...] = h_ref[...] * dis
    dis_ref[...] = jnp.broadcast_to(dis, (R, 8))

  return pl.pallas_call(
      body,
      grid=(N // R,),
      in_specs=[
          pl.BlockSpec((R, 128), lambda i: (i, 0)),
          pl.BlockSpec((2, R, 128), lambda i: (0, i, 0)),
      ],
      out_specs=[
          pl.BlockSpec((R, 128), lambda i: (i, 0)),
          pl.BlockSpec((R, 8), lambda i: (i, 0)),
      ],
      out_shape=[
          jax.ShapeDtypeStruct((N, 128), jnp.float32),
          jax.ShapeDtypeStruct((N, 8), jnp.float32),
      ],
  )(h_in, deg)


def _tc_b(agg1, u1, dis8, b_in, w1, R=1000):
  def body(ag_ref, u1_ref, dis_ref, bin_ref, w1_ref, h1_ref, u2_ref):
    dis = dis_ref[:, 0:1]
    h1 = dis * (ag_ref[0] + ag_ref[1] + u1_ref[...]) + bin_ref[...]
    h1 = jnp.maximum(h1, 0.0)
    h1_ref[...] = h1
    u2_ref[...] = jnp.dot(h1, w1_ref[...],
                          preferred_element_type=jnp.float32) * dis

  return pl.pallas_call(
      body,
      grid=(N // R,),
      in_specs=[
          pl.BlockSpec((2, R, 128), lambda i: (0, i, 0)),
          pl.BlockSpec((R, 128), lambda i: (i, 0)),
          pl.BlockSpec((R, 8), lambda i: (i, 0)),
          pl.BlockSpec((1, 128), lambda i: (0, 0)),
          pl.BlockSpec((128, 128), lambda i: (0, 0)),
      ],
      out_specs=[
          pl.BlockSpec((R, 128), lambda i: (i, 0)),
          pl.BlockSpec((R, 128), lambda i: (i, 0)),
      ],
      out_shape=[
          jax.ShapeDtypeStruct((N, 128), jnp.float32),
          jax.ShapeDtypeStruct((N, 128), jnp.float32),
      ],
  )(agg1, u1, dis8, b_in, w1)


def _lstm_cell(g, c_prev):
  i = jax.nn.sigmoid(g[:, 0:128])
  f = jax.nn.sigmoid(g[:, 128:256])
  gg = jnp.tanh(g[:, 256:384])
  o = jax.nn.sigmoid(g[:, 384:512])
  c = f * c_prev + i * gg
  return o * jnp.tanh(c), c


def _tc_c(agg2, u2, h1, dis8, b1,
          wihf, whhf, bf, wihb, whhb, bb, wf, wb, w_out, R=400):
  def body(ag_ref, u2_ref, h1_ref, dis_ref, b1_ref,
           wihf_ref, whhf_ref, bf_ref, wihb_ref, whhb_ref, bb_ref,
           wf_ref, wb_ref, wo_ref, u3_ref):
    dis = dis_ref[:, 0:1]
    h2 = dis * (ag_ref[0] + ag_ref[1] + u2_ref[...]) + b1_ref[...]
    h2 = jnp.maximum(h2, 0.0)
    h1 = h1_ref[...]

    def mm(a, b):
      return jnp.dot(a, b, preferred_element_type=jnp.float32)

    def mmb(a, b_ref):
      return jnp.dot(a.astype(jnp.bfloat16),
                     b_ref[...].astype(jnp.bfloat16),
                     preferred_element_type=jnp.float32)

    hf0, cf0 = _lstm_cell(mmb(h1, wihf_ref) + bf_ref[...], 0.0)
    hf1, _ = _lstm_cell(mmb(h2, wihf_ref) + mmb(hf0, whhf_ref)
                        + bf_ref[...], cf0)
    hb0, cb0 = _lstm_cell(mmb(h2, wihb_ref) + bb_ref[...], 0.0)
    hb1, _ = _lstm_cell(mmb(h1, wihb_ref) + mmb(hb0, whhb_ref)
                        + bb_ref[...], cb0)
    a0 = mm(hf0, wf_ref[...]) + mm(hb1, wb_ref[...])
    a1 = mm(hf1, wf_ref[...]) + mm(hb0, wb_ref[...])
    alpha = jax.nn.sigmoid(a0[:, 0:1] - a1[:, 0:1])
    h3 = alpha * h1 + (1.0 - alpha) * h2
    u3_ref[...] = mm(h3, wo_ref[...]) * dis

  return pl.pallas_call(
      body,
      grid=(N // R,),
      in_specs=[
          pl.BlockSpec((2, R, 128), lambda i: (0, i, 0)),
          pl.BlockSpec((R, 128), lambda i: (i, 0)),
          pl.BlockSpec((R, 128), lambda i: (i, 0)),
          pl.BlockSpec((R, 8), lambda i: (i, 0)),
          pl.BlockSpec((1, 128), lambda i: (0, 0)),
          pl.BlockSpec((128, 512), lambda i: (0, 0)),
          pl.BlockSpec((128, 512), lambda i: (0, 0)),
          pl.BlockSpec((1, 512), lambda i: (0, 0)),
          pl.BlockSpec((128, 512), lambda i: (0, 0)),
          pl.BlockSpec((128, 512), lambda i: (0, 0)),
          pl.BlockSpec((1, 512), lambda i: (0, 0)),
          pl.BlockSpec((128, 8), lambda i: (0, 0)),
          pl.BlockSpec((128, 8), lambda i: (0, 0)),
          pl.BlockSpec((128, 128), lambda i: (0, 0)),
      ],
      out_specs=pl.BlockSpec((R, 128), lambda i: (i, 0)),
      out_shape=jax.ShapeDtypeStruct((N, 128), jnp.float32),
  )(agg2, u2, h1, dis8, b1,
    wihf, whhf, bf, wihb, whhb, bb, wf, wb, w_out)


def _tc_d(agg3, u3, dis8, b_out, R=1000):
  def body(ag_ref, u3_ref, dis_ref, bo_ref, out_ref):
    dis = dis_ref[:, 0:1]
    full = dis * (ag_ref[0] + ag_ref[1] + u3_ref[...])
    out_ref[...] = full[:, 0:40] + bo_ref[...]

  return pl.pallas_call(
      body,
      grid=(N // R,),
      in_specs=[
          pl.BlockSpec((2, R, 128), lambda i: (0, i, 0)),
          pl.BlockSpec((R, 128), lambda i: (i, 0)),
          pl.BlockSpec((R, 8), lambda i: (i, 0)),
          pl.BlockSpec((1, 40), lambda i: (0, 0)),
      ],
      out_specs=pl.BlockSpec((R, 40), lambda i: (i, 0)),
      out_shape=jax.ShapeDtypeStruct((N, 40), jnp.float32),
  )(agg3, u3, dis8, b_out)


@jax.jit
def kernel(x, edge_index, params):
  src = edge_index[0]
  dst = edge_index[1]

  w_in = params["in_gc"]["W"]
  b_in = params["in_gc"]["b"][None, :]
  w1 = params["gc1"]["W"]
  b1 = params["gc1"]["b"][None, :]
  w_out = jnp.pad(params["out_gc"]["W"], ((0, 0), (0, 88)))
  b_out = params["out_gc"]["b"][None, :]
  lp = params["out_jk"]["lstm"]
  wihf = lp["Wih_f"].T
  whhf = lp["Whh_f"].T
  bf = (lp["bih_f"] + lp["bhh_f"])[None, :]
  wihb = lp["Wih_b"].T
  whhb = lp["Whh_b"].T
  bb = (lp["bih_b"] + lp["bhh_b"])[None, :]
  att = params["out_jk"]["att_W"]
  wf = jnp.pad(att[:, :128].T, ((0, 0), (0, 7)))
  wb = jnp.pad(att[:, 128:].T, ((0, 0), (0, 7)))

  ones128 = jnp.ones((K, 128), jnp.float32)
  zeros128 = jnp.zeros((ROWS_PER_TILE, 128), jnp.float32)

  h_in = _tc_a0(x, w_in)
  deg = _degree(dst, zeros128, ones128)
  u1, dis8 = _tc_a1(h_in, deg)
  agg1 = _scatter128(u1, src, dst, zeros128)
  h1, u2 = _tc_b(agg1, u1, dis8, b_in, w1)
  agg2 = _scatter128(u2, src, dst, zeros128)
  u3 = _tc_c(agg2, u2, h1, dis8, b1,
             wihf, whhf, bf, wihb, whhb, bb, wf, wb, w_out)
  agg3 = _scatter128(u3, src, dst, zeros128)
  return _tc_d(agg3, u3, dis8, b_out)


# speedup vs baseline: 18.6347x; 1.0175x over previous
import functools

import jax
import jax.numpy as jnp
from jax import lax
from jax.experimental import pallas as pl
from jax.experimental.pallas import tpu as pltpu
from jax.experimental.pallas import tpu_sc as plsc

N = 10000
NP = 10240
E = 320000
NC = 2
NS = 16
NW = NC * NS
EW = E // NW
K = 80
ROWS_PER_TILE = NP // NS


def _make_scatter(W):
  mesh = plsc.VectorSubcoreMesh(core_axis_name="c", subcore_axis_name="s")

  @functools.partial(
      pl.kernel,
      out_type=jax.ShapeDtypeStruct((NC, NP, W), jnp.float32),
      mesh=mesh,
      scratch_types=(
          [pltpu.VMEM((K,), jnp.int32)] * 4
          + [pltpu.VMEM((K,), jnp.int32)] * 4
          + [pltpu.VMEM((K, W), jnp.float32)] * 4
          + [pltpu.VMEM_SHARED((NP, W), jnp.float32)]
          + [pltpu.SemaphoreType.DMA] * 12
      ),
  )
  def scatter_kernel(u_hbm, src_hbm, dst_hbm, zeros_hbm, out_hbm, *scr):
    sidx = scr[0:4]
    didx = scr[4:8]
    rows = scr[8:12]
    acc = scr[12]
    sem_g = scr[13:17]
    sem_s = scr[17:21]
    sem_i = scr[21:25]
    c = lax.axis_index("c")
    s = lax.axis_index("s")
    stripe = pl.ds(s * ROWS_PER_TILE, ROWS_PER_TILE)
    base0 = (c * NS + s) * EW

    def issue_idx(i, q):
      pltpu.async_copy(src_hbm.at[pl.ds(base0 + i * K, K)], sidx[q], sem_i[q])
      pltpu.async_copy(dst_hbm.at[pl.ds(base0 + i * K, K)], didx[q], sem_i[q])

    def wait_idx(q):
      pltpu.make_async_copy(src_hbm.at[pl.ds(base0, K)], sidx[q],
                            sem_i[q]).wait()
      pltpu.make_async_copy(dst_hbm.at[pl.ds(base0, K)], didx[q],
                            sem_i[q]).wait()

    def issue_gather(q):
      pltpu.async_copy(u_hbm.at[sidx[q]], rows[q], sem_g[q])

    def wait_gather(q):
      pltpu.make_async_copy(u_hbm.at[sidx[q]], rows[q], sem_g[q]).wait()

    def issue_scatter(q):
      pltpu.async_copy(rows[q], acc.at[didx[q]], sem_s[q], add=True)

    def wait_scatter(q):
      pltpu.make_async_copy(rows[q], acc.at[didx[q]], sem_s[q]).wait()

    issue_idx(0, 0)
    issue_idx(1, 1)
    pltpu.sync_copy(zeros_hbm, acc.at[stripe])
    wait_idx(0)
    issue_gather(0)
    plsc.subcore_barrier()

    def body(j, carry):
      for k in range(4):
        q = k
        q2 = (k + 2) % 4
        q3 = (k + 1) % 4
        wait_gather(q)
        issue_scatter(q)
        if k < 2:
          @pl.when(j > 0)
          def _(q2=q2):
            wait_scatter(q2)
          issue_idx(4 * j + k + 2, q2)
        elif k == 2:
          wait_scatter(q2)
          issue_idx(4 * j + k + 2, q2)
        else:
          wait_scatter(q2)
          @pl.when(j < 30)
          def _(q2=q2):
            issue_idx(4 * j + k + 2, q2)
        wait_idx(q3)
        issue_gather(q3)
      return carry

    lax.fori_loop(0, 31, body, 0)
    wait_gather(0)
    issue_scatter(0)
    wait_scatter(0)
    wait_scatter(2)
    wait_scatter(3)

    plsc.subcore_barrier()
    pltpu.sync_copy(acc.at[stripe], out_hbm.at[c, stripe])

  return scatter_kernel


def _make_degree():
  mesh = plsc.VectorSubcoreMesh(core_axis_name="c", subcore_axis_name="s")

  @functools.partial(
      pl.kernel,
      out_type=jax.ShapeDtypeStruct((NC, NP, 128), jnp.float32),
      mesh=mesh,
      scratch_types=[
          pltpu.VMEM((K,), jnp.int32),
          pltpu.VMEM((K,), jnp.int32),
          pltpu.VMEM((K, 128), jnp.float32),
          pltpu.VMEM_SHARED((NP, 128), jnp.float32),
          pltpu.SemaphoreType.DMA,
          pltpu.SemaphoreType.DMA,
      ],
  )
  def degree_kernel(dst_hbm, zeros_hbm, ones_hbm, out_hbm,
                    didx_a, didx_b, ones_v, acc, sem_sa, sem_sb):
    c = lax.axis_index("c")
    s = lax.axis_index("s")
    stripe = pl.ds(s * ROWS_PER_TILE, ROWS_PER_TILE)
    base0 = (c * NS + s) * EW
    n_iters = EW // K
    pltpu.sync_copy(dst_hbm.at[pl.ds(base0, K)], didx_a)
    pltpu.sync_copy(dst_hbm.at[pl.ds(base0 + K, K)], didx_b)
    pltpu.sync_copy(ones_hbm, ones_v)
    pltpu.sync_copy(zeros_hbm, acc.at[stripe])
    plsc.subcore_barrier()

    pltpu.async_copy(ones_v, acc.at[didx_a], sem_sa, add=True)
    pltpu.async_copy(ones_v, acc.at[didx_b], sem_sb, add=True)

    def body(j, carry):
      pltpu.make_async_copy(ones_v, acc.at[didx_a], sem_sa).wait()
      pltpu.sync_copy(dst_hbm.at[pl.ds(base0 + (2 * j + 2) * K, K)], didx_a)
      pltpu.async_copy(ones_v, acc.at[didx_a], sem_sa, add=True)
      pltpu.make_async_copy(ones_v, acc.at[didx_b], sem_sb).wait()
      pltpu.sync_copy(dst_hbm.at[pl.ds(base0 + (2 * j + 3) * K, K)], didx_b)
      pltpu.async_copy(ones_v, acc.at[didx_b], sem_sb, add=True)
      return carry

    lax.fori_loop(0, 61, body, 0)
    pltpu.make_async_copy(ones_v, acc.at[didx_a], sem_sa).wait()
    pltpu.sync_copy(dst_hbm.at[pl.ds(base0 + (n_iters - 1) * K, K)], didx_a)
    pltpu.async_copy(ones_v, acc.at[didx_a], sem_sa, add=True)
    pltpu.make_async_copy(ones_v, acc.at[didx_a], sem_sa).wait()
    pltpu.make_async_copy(ones_v, acc.at[didx_b], sem_sb).wait()
    plsc.subcore_barrier()
    pltpu.sync_copy(acc.at[stripe], out_hbm.at[c, stripe])

  return degree_kernel


_scatter128 = _make_scatter(128)
_degree = _make_degree()


def _tc_a0(x, w_in, R=1000):
  def body(x_ref, w_ref, h_ref):
    h_ref[...] = jnp.dot(x_ref[...], w_ref[...],
                         preferred_element_type=jnp.float32)

  return pl.pallas_call(
      body,
      grid=(N // R,),
      in_specs=[
          pl.BlockSpec((R, 128), lambda i: (i, 0)),
          pl.BlockSpec((128, 128), lambda i: (0, 0)),
      ],
      out_specs=pl.BlockSpec((R, 128), lambda i: (i, 0)),
      out_shape=jax.ShapeDtypeStruct((N, 128), jnp.float32),
  )(x, w_in)


def _tc_a1(h_in, deg, R=1000):
  def body(h_ref, dg_ref, u1_ref, dis_ref):
    deg = dg_ref[0][:, 0:1] + dg_ref[1][:, 0:1] + 1.0
    dis = lax.rsqrt(deg)
    u1_ref[...] = h_ref[...] * dis
    dis_ref[...] = jnp.broadcast_to(dis, (R, 8))

  return pl.pallas_call(
      body,
      grid=(N // R,),
      in_specs=[
          pl.BlockSpec((R, 128), lambda i: (i, 0)),
          pl.BlockSpec((2, R, 128), lambda i: (0, i, 0)),
      ],
      out_specs=[
          pl.BlockSpec((R, 128), lambda i: (i, 0)),
          pl.BlockSpec((R, 8), lambda i: (i, 0)),
      ],
      out_shape=[
          jax.ShapeDtypeStruct((N, 128), jnp.float32),
          jax.ShapeDtypeStruct((N, 8), jnp.float32),
      ],
  )(h_in, deg)


def _tc_b(agg1, u1, dis8, b_in, w1, R=1000):
  def body(ag_ref, u1_ref, dis_ref, bin_ref, w1_ref, h1_ref, u2_ref):
    dis = dis_ref[:, 0:1]
    h1 = dis * (ag_ref[0] + ag_ref[1] + u1_ref[...]) + bin_ref[...]
    h1 = jnp.maximum(h1, 0.0)
    h1_ref[...] = h1
    u2_ref[...] = jnp.dot(h1, w1_ref[...],
                          preferred_element_type=jnp.float32) * dis

  return pl.pallas_call(
      body,
      grid=(N // R,),
      in_specs=[
          pl.BlockSpec((2, R, 128), lambda i: (0, i, 0)),
          pl.BlockSpec((R, 128), lambda i: (i, 0)),
          pl.BlockSpec((R, 8), lambda i: (i, 0)),
          pl.BlockSpec((1, 128), lambda i: (0, 0)),
          pl.BlockSpec((128, 128), lambda i: (0, 0)),
      ],
      out_specs=[
          pl.BlockSpec((R, 128), lambda i: (i, 0)),
          pl.BlockSpec((R, 128), lambda i: (i, 0)),
      ],
      out_shape=[
          jax.ShapeDtypeStruct((N, 128), jnp.float32),
          jax.ShapeDtypeStruct((N, 128), jnp.float32),
      ],
  )(agg1, u1, dis8, b_in, w1)


def _lstm_cell(g, c_prev):
  i = jax.nn.sigmoid(g[:, 0:128])
  f = jax.nn.sigmoid(g[:, 128:256])
  gg = jnp.tanh(g[:, 256:384])
  o = jax.nn.sigmoid(g[:, 384:512])
  c = f * c_prev + i * gg
  return o * jnp.tanh(c), c


def _tc_c(agg2, u2, h1, dis8, b1,
          wihf, whhf, bf, wihb, whhb, bb, wf, wb, w_out, R=1000):
  def body(ag_ref, u2_ref, h1_ref, dis_ref, b1_ref,
           wihf_ref, whhf_ref, bf_ref, wihb_ref, whhb_ref, bb_ref,
           wf_ref, wb_ref, wo_ref, u3_ref):
    dis = dis_ref[:, 0:1]
    h2 = dis * (ag_ref[0] + ag_ref[1] + u2_ref[...]) + b1_ref[...]
    h2 = jnp.maximum(h2, 0.0)
    h1 = h1_ref[...]

    def mm(a, b):
      return jnp.dot(a, b, preferred_element_type=jnp.float32)

    def mmb(a, b_ref):
      return jnp.dot(a.astype(jnp.bfloat16),
                     b_ref[...].astype(jnp.bfloat16),
                     preferred_element_type=jnp.float32)

    hf0, cf0 = _lstm_cell(mmb(h1, wihf_ref) + bf_ref[...], 0.0)
    hf1, _ = _lstm_cell(mmb(h2, wihf_ref) + mmb(hf0, whhf_ref)
                        + bf_ref[...], cf0)
    hb0, cb0 = _lstm_cell(mmb(h2, wihb_ref) + bb_ref[...], 0.0)
    hb1, _ = _lstm_cell(mmb(h1, wihb_ref) + mmb(hb0, whhb_ref)
                        + bb_ref[...], cb0)
    a0 = mm(hf0, wf_ref[...]) + mm(hb1, wb_ref[...])
    a1 = mm(hf1, wf_ref[...]) + mm(hb0, wb_ref[...])
    alpha = jax.nn.sigmoid(a0[:, 0:1] - a1[:, 0:1])
    h3 = alpha * h1 + (1.0 - alpha) * h2
    u3_ref[...] = mm(h3, wo_ref[...]) * dis

  return pl.pallas_call(
      body,
      grid=(N // R,),
      in_specs=[
          pl.BlockSpec((2, R, 128), lambda i: (0, i, 0)),
          pl.BlockSpec((R, 128), lambda i: (i, 0)),
          pl.BlockSpec((R, 128), lambda i: (i, 0)),
          pl.BlockSpec((R, 8), lambda i: (i, 0)),
          pl.BlockSpec((1, 128), lambda i: (0, 0)),
          pl.BlockSpec((128, 512), lambda i: (0, 0)),
          pl.BlockSpec((128, 512), lambda i: (0, 0)),
          pl.BlockSpec((1, 512), lambda i: (0, 0)),
          pl.BlockSpec((128, 512), lambda i: (0, 0)),
          pl.BlockSpec((128, 512), lambda i: (0, 0)),
          pl.BlockSpec((1, 512), lambda i: (0, 0)),
          pl.BlockSpec((128, 8), lambda i: (0, 0)),
          pl.BlockSpec((128, 8), lambda i: (0, 0)),
          pl.BlockSpec((128, 128), lambda i: (0, 0)),
      ],
      out_specs=pl.BlockSpec((R, 128), lambda i: (i, 0)),
      out_shape=jax.ShapeDtypeStruct((N, 128), jnp.float32),
  )(agg2, u2, h1, dis8, b1,
    wihf, whhf, bf, wihb, whhb, bb, wf, wb, w_out)


def _tc_d(agg3, u3, dis8, b_out, R=1000):
  def body(ag_ref, u3_ref, dis_ref, bo_ref, out_ref):
    dis = dis_ref[:, 0:1]
    full = dis * (ag_ref[0] + ag_ref[1] + u3_ref[...])
    out_ref[...] = full[:, 0:40] + bo_ref[...]

  return pl.pallas_call(
      body,
      grid=(N // R,),
      in_specs=[
          pl.BlockSpec((2, R, 128), lambda i: (0, i, 0)),
          pl.BlockSpec((R, 128), lambda i: (i, 0)),
          pl.BlockSpec((R, 8), lambda i: (i, 0)),
          pl.BlockSpec((1, 40), lambda i: (0, 0)),
      ],
      out_specs=pl.BlockSpec((R, 40), lambda i: (i, 0)),
      out_shape=jax.ShapeDtypeStruct((N, 40), jnp.float32),
  )(agg3, u3, dis8, b_out)


@jax.jit
def kernel(x, edge_index, params):
  src = edge_index[0]
  dst = edge_index[1]

  w_in = params["in_gc"]["W"]
  b_in = params["in_gc"]["b"][None, :]
  w1 = params["gc1"]["W"]
  b1 = params["gc1"]["b"][None, :]
  w_out = jnp.pad(params["out_gc"]["W"], ((0, 0), (0, 88)))
  b_out = params["out_gc"]["b"][None, :]
  lp = params["out_jk"]["lstm"]
  wihf = lp["Wih_f"].T
  whhf = lp["Whh_f"].T
  bf = (lp["bih_f"] + lp["bhh_f"])[None, :]
  wihb = lp["Wih_b"].T
  whhb = lp["Whh_b"].T
  bb = (lp["bih_b"] + lp["bhh_b"])[None, :]
  att = params["out_jk"]["att_W"]
  wf = jnp.pad(att[:, :128].T, ((0, 0), (0, 7)))
  wb = jnp.pad(att[:, 128:].T, ((0, 0), (0, 7)))

  ones128 = jnp.ones((K, 128), jnp.float32)
  zeros128 = jnp.zeros((ROWS_PER_TILE, 128), jnp.float32)

  h_in = _tc_a0(x, w_in)
  deg = _degree(dst, zeros128, ones128)
  u1, dis8 = _tc_a1(h_in, deg)
  agg1 = _scatter128(u1, src, dst, zeros128)
  h1, u2 = _tc_b(agg1, u1, dis8, b_in, w1)
  agg2 = _scatter128(u2, src, dst, zeros128)
  u3 = _tc_c(agg2, u2, h1, dis8, b1,
             wihf, whhf, bf, wihb, whhb, bb, wf, wb, w_out)
  agg3 = _scatter128(u3, src, dst, zeros128)
  return _tc_d(agg3, u3, dis8, b_out)
